# Initial kernel scaffold; baseline (speedup 1.0000x reference)
import jax
import jax.numpy as jnp
from jax.experimental import pallas as pl


def kernel(x, edge_index, w1, w2, dec1_w, dec1_b, dec2_w, dec2_b):
    raise NotImplementedError("write your pallas kernel here")


# speedup vs baseline: 36.2197x; 36.2197x over previous
import functools

import jax
import jax.numpy as jnp
from jax import lax
from jax.experimental import pallas as pl
from jax.experimental.pallas import tpu as pltpu
from jax.experimental.pallas import tpu_sc as plsc

N = 100000
E = 3200000
D = 16
PRED_H = 32
ALPHA = 0.5
import numpy as _np
BETA1 = float(_np.log(1.0 / 1.0 + 1.0))
BETA2 = float(_np.log(1.0 / 2.0 + 1.0))

NC = 2
NS = 16
NW = NC * NS
LANES = 128

NP = 100352
RPT = NP // NS
EP = 3211264
EROWS = EP // LANES
ROWS_PER_W = EROWS // NW
KD = 16
NGROUPS_D = ROWS_PER_W // KD
KM = 8
NGROUPS_M = ROWS_PER_W // KM

RB = 1024
G = NP // RB

_f32 = jnp.float32
_i32 = jnp.int32


def _fill(ref, n, value):
    def body(i, _):
        ref[pl.ds(i * 16, 16)] = jnp.full((16,), value, _f32)
        return 0
    lax.fori_loop(0, n // 16, body, 0)


def _sc_degrees_body(src_hbm, dst_hbm, dego_hbm, degi_hbm,
                     sidx, didx, ones_v, zbuf, dego_s, degi_s, semo, semi):
    cid = lax.axis_index("c")
    sid = lax.axis_index("s")
    wid = sid * NC + cid

    _fill(ones_v, LANES, 1.0)
    _fill(zbuf, RPT, 0.0)
    pltpu.sync_copy(zbuf, dego_s.at[pl.ds(sid * RPT, RPT)])
    pltpu.sync_copy(zbuf, degi_s.at[pl.ds(sid * RPT, RPT)])
    plsc.subcore_barrier()

    row0 = wid * ROWS_PER_W

    def group(g, _):
        base = row0 + g * KD
        pltpu.sync_copy(src_hbm.at[pl.ds(base, KD)], sidx)
        pltpu.sync_copy(dst_hbm.at[pl.ds(base, KD)], didx)
        descs = []
        for j in range(KD):
            descs.append(pltpu.async_copy(
                ones_v, dego_s.at[sidx.at[j]], semo, add=True))
            descs.append(pltpu.async_copy(
                ones_v, degi_s.at[didx.at[j]], semi, add=True))
        for d in descs:
            d.wait()
        return 0

    lax.fori_loop(0, NGROUPS_D, group, 0)
    plsc.subcore_barrier()

    off = cid * NP + sid * RPT
    pltpu.sync_copy(dego_s.at[pl.ds(sid * RPT, RPT)], dego_hbm.at[pl.ds(off, RPT)])
    pltpu.sync_copy(degi_s.at[pl.ds(sid * RPT, RPT)], degi_hbm.at[pl.ds(off, RPT)])


def _sc_message_body(y_hbm, src_hbm, dst_hbm, agg_hbm,
                     sidx, didx, rows, agg_s, semg, sems):
    cid = lax.axis_index("c")
    sid = lax.axis_index("s")
    wid = sid * NC + cid

    def zrow(i, _):
        rows[i, :] = jnp.zeros((D,), _f32)
        return 0
    lax.fori_loop(0, KM * LANES, zrow, 0)
    r0 = sid * RPT
    for q in range(RPT // (KM * LANES)):
        pltpu.sync_copy(rows, agg_s.at[pl.ds(r0 + q * KM * LANES, KM * LANES)])
    rem = RPT % (KM * LANES)
    if rem:
        pltpu.sync_copy(rows.at[pl.ds(0, rem)],
                        agg_s.at[pl.ds(r0 + RPT - rem, rem)])
    plsc.subcore_barrier()

    row0 = wid * ROWS_PER_W

    def group(g, _):
        base = row0 + g * KM
        pltpu.sync_copy(src_hbm.at[pl.ds(base, KM)], sidx)
        pltpu.sync_copy(dst_hbm.at[pl.ds(base, KM)], didx)
        gd = []
        for j in range(KM):
            gd.append(pltpu.async_copy(
                y_hbm.at[sidx.at[j]], rows.at[pl.ds(j * LANES, LANES)], semg))
        for d in gd:
            d.wait()
        sd = []
        for j in range(KM):
            sd.append(pltpu.async_copy(
                rows.at[pl.ds(j * LANES, LANES)], agg_s.at[didx.at[j]],
                sems, add=True))
        for d in sd:
            d.wait()
        return 0

    lax.fori_loop(0, NGROUPS_M, group, 0)
    plsc.subcore_barrier()

    off = cid * NP + sid * RPT
    pltpu.sync_copy(agg_s.at[pl.ds(sid * RPT, RPT)], agg_hbm.at[pl.ds(off, RPT)])


@functools.lru_cache(maxsize=None)
def _sc_kernels():
    mesh = plsc.VectorSubcoreMesh(
        core_axis_name="c", subcore_axis_name="s",
        num_cores=NC, num_subcores=NS)
    params = pltpu.CompilerParams(use_tc_tiling_on_sc=False)
    sc_degrees = pl.kernel(
        _sc_degrees_body,
        out_type=(
            jax.ShapeDtypeStruct((NC * NP,), _f32),
            jax.ShapeDtypeStruct((NC * NP,), _f32),
        ),
        mesh=mesh,
        scratch_types=[
            pltpu.VMEM((KD, LANES), _i32),
            pltpu.VMEM((KD, LANES), _i32),
            pltpu.VMEM((LANES,), _f32),
            pltpu.VMEM((RPT,), _f32),
            pltpu.VMEM_SHARED((NP,), _f32),
            pltpu.VMEM_SHARED((NP,), _f32),
            pltpu.SemaphoreType.DMA,
            pltpu.SemaphoreType.DMA,
        ],
        compiler_params=params,
    )
    sc_message = pl.kernel(
        _sc_message_body,
        out_type=jax.ShapeDtypeStruct((NC * NP, D), _f32),
        mesh=mesh,
        scratch_types=[
            pltpu.VMEM((KM, LANES), _i32),
            pltpu.VMEM((KM, LANES), _i32),
            pltpu.VMEM((KM * LANES, D), _f32),
            pltpu.VMEM_SHARED((NP, D), _f32),
            pltpu.SemaphoreType.DMA,
            pltpu.SemaphoreType.DMA,
        ],
        compiler_params=params,
    )
    return sc_degrees, sc_message


def _scale_col(v_1xr):
    return jnp.reshape(v_1xr, (RB, 1))


def _tc_prep_body(x_ref, do0_ref, do1_ref, y_ref):
    s_out = lax.rsqrt(do0_ref[0] + do1_ref[0] + 1.0)
    y_ref[...] = x_ref[...] * _scale_col(s_out)


def _tc_combine1_body(p0_ref, p1_ref, yc_ref, x0_ref,
                      di0_ref, di1_ref, do0_ref, do1_ref, w_ref, y2_ref):
    s_in = lax.rsqrt(di0_ref[0] + di1_ref[0] + 1.0)
    s_out = lax.rsqrt(do0_ref[0] + do1_ref[0] + 1.0)
    agg = (p0_ref[...] + p1_ref[...] + yc_ref[...]) * _scale_col(s_in)
    h = (1.0 - ALPHA) * agg + ALPHA * x0_ref[...]
    hw = jnp.dot(h, w_ref[...], preferred_element_type=_f32)
    x1 = jnp.maximum((1.0 - BETA1) * h + BETA1 * hw, 0.0)
    y2_ref[...] = x1 * _scale_col(s_out)


def _tc_combine2_body(p0_ref, p1_ref, yc_ref, x0_ref,
                      di0_ref, di1_ref, w_ref,
                      d1w_ref, d1b_ref, d2wt_ref, d2b_ref, o_ref, acc_ref):
    i = pl.program_id(0)

    @pl.when(i == 0)
    def _():
        acc_ref[...] = jnp.zeros((1, D), _f32)

    s_in = lax.rsqrt(di0_ref[0] + di1_ref[0] + 1.0)
    agg = (p0_ref[...] + p1_ref[...] + yc_ref[...]) * _scale_col(s_in)
    h = (1.0 - ALPHA) * agg + ALPHA * x0_ref[...]
    hw = jnp.dot(h, w_ref[...], preferred_element_type=_f32)
    x2 = jnp.maximum((1.0 - BETA2) * h + BETA2 * hw, 0.0)
    acc_ref[...] += jnp.sum(x2, axis=0, keepdims=True)

    @pl.when(i == G - 1)
    def _():
        pooled = acc_ref[...] * (1.0 / N)
        hm = jnp.dot(pooled, d1w_ref[...], preferred_element_type=_f32)
        hm = jnp.maximum(hm + d1b_ref[...], 0.0)
        z = jnp.sum(hm * d2wt_ref[...], axis=1, keepdims=True) + d2b_ref[...]
        o_ref[...] = 1.0 / (1.0 + jnp.exp(-z))


_row_spec = pl.BlockSpec((RB, D), lambda i: (i, 0))
_vec_spec = pl.BlockSpec((1, 1, RB), lambda i: (i, 0, 0))


def _tc_prep(x_pad, dego0, dego1):
    return pl.pallas_call(
        _tc_prep_body,
        grid=(G,),
        in_specs=[_row_spec, _vec_spec, _vec_spec],
        out_specs=_row_spec,
        out_shape=jax.ShapeDtypeStruct((NP, D), _f32),
    )(x_pad, dego0, dego1)


def _tc_combine1(p0, p1, ycur, x0, degi0, degi1, dego0, dego1, w):
    wspec = pl.BlockSpec((D, D), lambda i: (0, 0))
    return pl.pallas_call(
        _tc_combine1_body,
        grid=(G,),
        in_specs=[_row_spec, _row_spec, _row_spec, _row_spec,
                  _vec_spec, _vec_spec, _vec_spec, _vec_spec, wspec],
        out_specs=_row_spec,
        out_shape=jax.ShapeDtypeStruct((NP, D), _f32),
    )(p0, p1, ycur, x0, degi0, degi1, dego0, dego1, w)


def _tc_combine2(p0, p1, ycur, x0, degi0, degi1, w, d1w, d1b, d2wt, d2b):
    return pl.pallas_call(
        _tc_combine2_body,
        grid=(G,),
        in_specs=[_row_spec, _row_spec, _row_spec, _row_spec,
                  _vec_spec, _vec_spec,
                  pl.BlockSpec((D, D), lambda i: (0, 0)),
                  pl.BlockSpec((D, PRED_H), lambda i: (0, 0)),
                  pl.BlockSpec((1, PRED_H), lambda i: (0, 0)),
                  pl.BlockSpec((1, PRED_H), lambda i: (0, 0)),
                  pl.BlockSpec((1, 1), lambda i: (0, 0))],
        out_specs=pl.BlockSpec((1, 1), lambda i: (0, 0)),
        out_shape=jax.ShapeDtypeStruct((1, 1), _f32),
        scratch_shapes=[pltpu.VMEM((1, D), _f32)],
    )(p0, p1, ycur, x0, degi0, degi1, w, d1w, d1b, d2wt, d2b)


def kernel(x, edge_index, w1, w2, dec1_w, dec1_b, dec2_w, dec2_b):
    src = edge_index[0]
    dst = edge_index[1]
    pad_idx = jnp.full((EP - E,), N, dtype=_i32)
    src_p = jnp.concatenate([src, pad_idx]).reshape(EROWS, LANES)
    dst_p = jnp.concatenate([dst, pad_idx]).reshape(EROWS, LANES)
    x_pad = jnp.concatenate([x, jnp.zeros((NP - N, D), _f32)], axis=0)

    sc_degrees, sc_message = _sc_kernels()
    dego_f, degi_f = sc_degrees(src_p, dst_p)
    dego0 = dego_f[:NP].reshape(G, 1, RB)
    dego1 = dego_f[NP:].reshape(G, 1, RB)
    degi0 = degi_f[:NP].reshape(G, 1, RB)
    degi1 = degi_f[NP:].reshape(G, 1, RB)

    y1 = _tc_prep(x_pad, dego0, dego1)

    agg1 = sc_message(y1, src_p, dst_p)
    a1p0, a1p1 = agg1[:NP], agg1[NP:]

    y2 = _tc_combine1(a1p0, a1p1, y1, x_pad, degi0, degi1, dego0, dego1, w1)

    agg2 = sc_message(y2, src_p, dst_p)
    a2p0, a2p1 = agg2[:NP], agg2[NP:]

    o = _tc_combine2(a2p0, a2p1, y2, x_pad, degi0, degi1, w2,
                     dec1_w, dec1_b.reshape(1, PRED_H),
                     dec2_w.reshape(1, PRED_H), dec2_b.reshape(1, 1))
    return o


# speedup vs baseline: 41.3111x; 1.1406x over previous
import functools

import jax
import jax.numpy as jnp
from jax import lax
from jax.experimental import pallas as pl
from jax.experimental.pallas import tpu as pltpu
from jax.experimental.pallas import tpu_sc as plsc

N = 100000
E = 3200000
D = 16
PRED_H = 32
ALPHA = 0.5
import numpy as _np
BETA1 = float(_np.log(1.0 / 1.0 + 1.0))
BETA2 = float(_np.log(1.0 / 2.0 + 1.0))

NC = 2
NS = 16
NW = NC * NS
LANES = 128

NP = 100352
RPT = NP // NS
EP = 3211264
EROWS = EP // LANES
ROWS_PER_W = EROWS // NW
KD = 16
NGROUPS_D = ROWS_PER_W // KD
KM = 4
NGROUPS_M = ROWS_PER_W // KM

RB = 1024
G = NP // RB

_f32 = jnp.float32
_i32 = jnp.int32


def _fill(ref, n, value):
    def body(i, _):
        ref[pl.ds(i * 16, 16)] = jnp.full((16,), value, _f32)
        return 0
    lax.fori_loop(0, n // 16, body, 0)


def _sc_degrees_body(src_hbm, dst_hbm, dego_hbm, degi_hbm,
                     sidx, didx, ones_v, zbuf, dego_s, degi_s, semo, semi):
    cid = lax.axis_index("c")
    sid = lax.axis_index("s")
    wid = sid * NC + cid

    _fill(ones_v, LANES, 1.0)
    _fill(zbuf, RPT, 0.0)
    pltpu.sync_copy(zbuf, dego_s.at[pl.ds(sid * RPT, RPT)])
    pltpu.sync_copy(zbuf, degi_s.at[pl.ds(sid * RPT, RPT)])
    plsc.subcore_barrier()

    row0 = wid * ROWS_PER_W

    def group(g, _):
        base = row0 + g * KD
        pltpu.sync_copy(src_hbm.at[pl.ds(base, KD)], sidx)
        pltpu.sync_copy(dst_hbm.at[pl.ds(base, KD)], didx)
        descs = []
        for j in range(KD):
            descs.append(pltpu.async_copy(
                ones_v, dego_s.at[sidx.at[j]], semo, add=True))
            descs.append(pltpu.async_copy(
                ones_v, degi_s.at[didx.at[j]], semi, add=True))
        for d in descs:
            d.wait()
        return 0

    lax.fori_loop(0, NGROUPS_D, group, 0)
    plsc.subcore_barrier()

    off = cid * NP + sid * RPT
    pltpu.sync_copy(dego_s.at[pl.ds(sid * RPT, RPT)], dego_hbm.at[pl.ds(off, RPT)])
    pltpu.sync_copy(degi_s.at[pl.ds(sid * RPT, RPT)], degi_hbm.at[pl.ds(off, RPT)])


def _sc_message_body(y_hbm, src_hbm, dst_hbm, agg_hbm,
                     sidx, didx, rows, agg_s, semi, semg, sems):
    cid = lax.axis_index("c")
    sid = lax.axis_index("s")
    wid = sid * NC + cid
    GROUP_ROWS = KM * LANES

    def zrow(i, _):
        rows[0, i, :] = jnp.zeros((D,), _f32)
        return 0
    lax.fori_loop(0, GROUP_ROWS, zrow, 0)
    r0 = sid * RPT
    for q in range(RPT // GROUP_ROWS):
        pltpu.sync_copy(rows.at[0], agg_s.at[pl.ds(r0 + q * GROUP_ROWS, GROUP_ROWS)])
    rem = RPT % GROUP_ROWS
    if rem:
        pltpu.sync_copy(rows.at[0, pl.ds(0, rem)],
                        agg_s.at[pl.ds(r0 + RPT - rem, rem)])
    plsc.subcore_barrier()

    row0 = wid * ROWS_PER_W

    def fire_idx(g):
        base = row0 + g * KM
        pltpu.async_copy(src_hbm.at[pl.ds(base, KM)], sidx.at[g % 2], semi)
        pltpu.async_copy(dst_hbm.at[pl.ds(base, KM)], didx.at[g % 3], semi)

    def drain_idx():
        for _ in range(2):
            pltpu.make_async_copy(
                src_hbm.at[pl.ds(0, KM)], sidx.at[0], semi).wait()

    def drain_scatter():
        for _ in range(KM):
            pltpu.make_async_copy(
                y_hbm.at[pl.ds(0, LANES)], rows.at[0, pl.ds(0, LANES)],
                sems).wait()

    fire_idx(0)

    def group(g, _):
        br = g % 2
        bd = g % 3

        @pl.when(g >= 2)
        def _():
            drain_scatter()

        drain_idx()

        @pl.when(g + 1 < NGROUPS_M)
        def _():
            fire_idx(g + 1)

        gd = []
        for j in range(KM):
            gd.append(pltpu.async_copy(
                y_hbm.at[sidx.at[br, j]],
                rows.at[br, pl.ds(j * LANES, LANES)], semg))
        for d in gd:
            d.wait()
        for j in range(KM):
            pltpu.async_copy(
                rows.at[br, pl.ds(j * LANES, LANES)], agg_s.at[didx.at[bd, j]],
                sems, add=True)
        return 0

    lax.fori_loop(0, NGROUPS_M, group, 0)
    drain_scatter()
    drain_scatter()
    plsc.subcore_barrier()

    off = cid * NP + sid * RPT
    pltpu.sync_copy(agg_s.at[pl.ds(sid * RPT, RPT)], agg_hbm.at[pl.ds(off, RPT)])


@functools.lru_cache(maxsize=None)
def _sc_kernels():
    mesh = plsc.VectorSubcoreMesh(
        core_axis_name="c", subcore_axis_name="s",
        num_cores=NC, num_subcores=NS)
    params = pltpu.CompilerParams(use_tc_tiling_on_sc=False)
    sc_degrees = pl.kernel(
        _sc_degrees_body,
        out_type=(
            jax.ShapeDtypeStruct((NC * NP,), _f32),
            jax.ShapeDtypeStruct((NC * NP,), _f32),
        ),
        mesh=mesh,
        scratch_types=[
            pltpu.VMEM((KD, LANES), _i32),
            pltpu.VMEM((KD, LANES), _i32),
            pltpu.VMEM((LANES,), _f32),
            pltpu.VMEM((RPT,), _f32),
            pltpu.VMEM_SHARED((NP,), _f32),
            pltpu.VMEM_SHARED((NP,), _f32),
            pltpu.SemaphoreType.DMA,
            pltpu.SemaphoreType.DMA,
        ],
        compiler_params=params,
    )
    sc_message = pl.kernel(
        _sc_message_body,
        out_type=jax.ShapeDtypeStruct((NC * NP, D), _f32),
        mesh=mesh,
        scratch_types=[
            pltpu.VMEM((2, KM, LANES), _i32),
            pltpu.VMEM((3, KM, LANES), _i32),
            pltpu.VMEM((2, KM * LANES, D), _f32),
            pltpu.VMEM_SHARED((NP, D), _f32),
            pltpu.SemaphoreType.DMA,
            pltpu.SemaphoreType.DMA,
            pltpu.SemaphoreType.DMA,
        ],
        compiler_params=params,
    )
    return sc_degrees, sc_message


def _scale_col(v_1xr):
    return jnp.reshape(v_1xr, (RB, 1))


def _tc_prep_body(x_ref, do0_ref, do1_ref, y_ref):
    s_out = lax.rsqrt(do0_ref[0] + do1_ref[0] + 1.0)
    y_ref[...] = x_ref[...] * _scale_col(s_out)


def _tc_combine1_body(p0_ref, p1_ref, yc_ref, x0_ref,
                      di0_ref, di1_ref, do0_ref, do1_ref, w_ref, y2_ref):
    s_in = lax.rsqrt(di0_ref[0] + di1_ref[0] + 1.0)
    s_out = lax.rsqrt(do0_ref[0] + do1_ref[0] + 1.0)
    agg = (p0_ref[...] + p1_ref[...] + yc_ref[...]) * _scale_col(s_in)
    h = (1.0 - ALPHA) * agg + ALPHA * x0_ref[...]
    hw = jnp.dot(h, w_ref[...], preferred_element_type=_f32)
    x1 = jnp.maximum((1.0 - BETA1) * h + BETA1 * hw, 0.0)
    y2_ref[...] = x1 * _scale_col(s_out)


def _tc_combine2_body(p0_ref, p1_ref, yc_ref, x0_ref,
                      di0_ref, di1_ref, w_ref,
                      d1w_ref, d1b_ref, d2wt_ref, d2b_ref, o_ref, acc_ref):
    i = pl.program_id(0)

    @pl.when(i == 0)
    def _():
        acc_ref[...] = jnp.zeros((1, D), _f32)

    s_in = lax.rsqrt(di0_ref[0] + di1_ref[0] + 1.0)
    agg = (p0_ref[...] + p1_ref[...] + yc_ref[...]) * _scale_col(s_in)
    h = (1.0 - ALPHA) * agg + ALPHA * x0_ref[...]
    hw = jnp.dot(h, w_ref[...], preferred_element_type=_f32)
    x2 = jnp.maximum((1.0 - BETA2) * h + BETA2 * hw, 0.0)
    acc_ref[...] += jnp.sum(x2, axis=0, keepdims=True)

    @pl.when(i == G - 1)
    def _():
        pooled = acc_ref[...] * (1.0 / N)
        hm = jnp.dot(pooled, d1w_ref[...], preferred_element_type=_f32)
        hm = jnp.maximum(hm + d1b_ref[...], 0.0)
        z = jnp.sum(hm * d2wt_ref[...], axis=1, keepdims=True) + d2b_ref[...]
        o_ref[...] = 1.0 / (1.0 + jnp.exp(-z))


_row_spec = pl.BlockSpec((RB, D), lambda i: (i, 0))
_vec_spec = pl.BlockSpec((1, 1, RB), lambda i: (i, 0, 0))


def _tc_prep(x_pad, dego0, dego1):
    return pl.pallas_call(
        _tc_prep_body,
        grid=(G,),
        in_specs=[_row_spec, _vec_spec, _vec_spec],
        out_specs=_row_spec,
        out_shape=jax.ShapeDtypeStruct((NP, D), _f32),
    )(x_pad, dego0, dego1)


def _tc_combine1(p0, p1, ycur, x0, degi0, degi1, dego0, dego1, w):
    wspec = pl.BlockSpec((D, D), lambda i: (0, 0))
    return pl.pallas_call(
        _tc_combine1_body,
        grid=(G,),
        in_specs=[_row_spec, _row_spec, _row_spec, _row_spec,
                  _vec_spec, _vec_spec, _vec_spec, _vec_spec, wspec],
        out_specs=_row_spec,
        out_shape=jax.ShapeDtypeStruct((NP, D), _f32),
    )(p0, p1, ycur, x0, degi0, degi1, dego0, dego1, w)


def _tc_combine2(p0, p1, ycur, x0, degi0, degi1, w, d1w, d1b, d2wt, d2b):
    return pl.pallas_call(
        _tc_combine2_body,
        grid=(G,),
        in_specs=[_row_spec, _row_spec, _row_spec, _row_spec,
                  _vec_spec, _vec_spec,
                  pl.BlockSpec((D, D), lambda i: (0, 0)),
                  pl.BlockSpec((D, PRED_H), lambda i: (0, 0)),
                  pl.BlockSpec((1, PRED_H), lambda i: (0, 0)),
                  pl.BlockSpec((1, PRED_H), lambda i: (0, 0)),
                  pl.BlockSpec((1, 1), lambda i: (0, 0))],
        out_specs=pl.BlockSpec((1, 1), lambda i: (0, 0)),
        out_shape=jax.ShapeDtypeStruct((1, 1), _f32),
        scratch_shapes=[pltpu.VMEM((1, D), _f32)],
    )(p0, p1, ycur, x0, degi0, degi1, w, d1w, d1b, d2wt, d2b)


def kernel(x, edge_index, w1, w2, dec1_w, dec1_b, dec2_w, dec2_b):
    src = edge_index[0]
    dst = edge_index[1]
    pad_idx = jnp.full((EP - E,), N, dtype=_i32)
    src_p = jnp.concatenate([src, pad_idx]).reshape(EROWS, LANES)
    dst_p = jnp.concatenate([dst, pad_idx]).reshape(EROWS, LANES)
    x_pad = jnp.concatenate([x, jnp.zeros((NP - N, D), _f32)], axis=0)

    sc_degrees, sc_message = _sc_kernels()
    dego_f, degi_f = sc_degrees(src_p, dst_p)
    dego0 = dego_f[:NP].reshape(G, 1, RB)
    dego1 = dego_f[NP:].reshape(G, 1, RB)
    degi0 = degi_f[:NP].reshape(G, 1, RB)
    degi1 = degi_f[NP:].reshape(G, 1, RB)

    y1 = _tc_prep(x_pad, dego0, dego1)

    agg1 = sc_message(y1, src_p, dst_p)
    a1p0, a1p1 = agg1[:NP], agg1[NP:]

    y2 = _tc_combine1(a1p0, a1p1, y1, x_pad, degi0, degi1, dego0, dego1, w1)

    agg2 = sc_message(y2, src_p, dst_p)
    a2p0, a2p1 = agg2[:NP], agg2[NP:]

    o = _tc_combine2(a2p0, a2p1, y2, x_pad, degi0, degi1, w2,
                     dec1_w, dec1_b.reshape(1, PRED_H),
                     dec2_w.reshape(1, PRED_H), dec2_b.reshape(1, 1))
    return o


# speedup vs baseline: 65.1303x; 1.5766x over previous
import functools

import jax
import jax.numpy as jnp
import numpy as _np
from jax import lax
from jax.experimental import pallas as pl
from jax.experimental.pallas import tpu as pltpu
from jax.experimental.pallas import tpu_sc as plsc

N = 100000
E = 3200000
D = 16
PRED_H = 32
ALPHA = 0.5
BETA1 = float(_np.log(1.0 / 1.0 + 1.0))
BETA2 = float(_np.log(1.0 / 2.0 + 1.0))

NC = 2
NS = 16
NW = NC * NS
LANES = 128

NP = 100352
RPT = NP // NS
EROWS = E // LANES
KG = 4
GTOT = EROWS // KG
GBASE = GTOT // NW
GREM = GTOT % NW

PR = NP // 8
RB = 1024
PB = RB // 8
G = NP // RB

_f32 = jnp.float32
_i32 = jnp.int32


def _fill(ref, n, value):
    def body(i, _):
        ref[pl.ds(i * 16, 16)] = jnp.full((16,), value, _f32)
        return 0
    lax.fori_loop(0, n // 16, body, 0)


def _worker_groups(wid):
    extra = jnp.minimum(wid, GREM)
    return wid * GBASE + extra, GBASE + jnp.where(wid < GREM, 1, 0)


def _sc_degrees_body(src_hbm, dst_hbm, dego_hbm, degi_hbm,
                     sidx, didx, ones_v, zbuf, dego_s, degi_s, semi, semd):
    cid = lax.axis_index("c")
    sid = lax.axis_index("s")
    wid = sid * NC + cid

    _fill(ones_v, LANES, 1.0)
    _fill(zbuf, RPT, 0.0)
    pltpu.sync_copy(zbuf, dego_s.at[pl.ds(sid * RPT, RPT)])
    pltpu.sync_copy(zbuf, degi_s.at[pl.ds(sid * RPT, RPT)])
    plsc.subcore_barrier()

    g0, ng = _worker_groups(wid)

    def fire_idx(g):
        base = (g0 + g) * KG
        pltpu.async_copy(src_hbm.at[pl.ds(base, KG)], sidx.at[g % 3], semi)
        pltpu.async_copy(dst_hbm.at[pl.ds(base, KG)], didx.at[g % 3], semi)

    def drain_idx():
        for _ in range(2):
            pltpu.make_async_copy(
                src_hbm.at[pl.ds(0, KG)], sidx.at[0], semi).wait()

    def drain_scatter():
        for _ in range(2 * KG):
            pltpu.make_async_copy(
                src_hbm.at[pl.ds(0, 1)], sidx.at[0, pl.ds(0, 1)], semd).wait()

    fire_idx(0)

    def group(g, _):
        b = g % 3

        @pl.when(g >= 2)
        def _():
            drain_scatter()

        drain_idx()

        @pl.when(g + 1 < ng)
        def _():
            fire_idx(g + 1)

        for j in range(KG):
            pltpu.async_copy(ones_v, dego_s.at[sidx.at[b, j]], semd, add=True)
            pltpu.async_copy(ones_v, degi_s.at[didx.at[b, j]], semd, add=True)
        return 0

    lax.fori_loop(0, ng, group, 0)
    drain_scatter()
    drain_scatter()
    plsc.subcore_barrier()

    off = cid * NP + sid * RPT
    pltpu.sync_copy(dego_s.at[pl.ds(sid * RPT, RPT)], dego_hbm.at[pl.ds(off, RPT)])
    pltpu.sync_copy(degi_s.at[pl.ds(sid * RPT, RPT)], degi_hbm.at[pl.ds(off, RPT)])


def _sc_message_body(y_hbm, src_hbm, dst_hbm, agg_hbm,
                     sidx, didx, rows, agg_s, semi, semg, sems):
    cid = lax.axis_index("c")
    sid = lax.axis_index("s")
    wid = sid * NC + cid
    GROUP_ROWS = KG * LANES

    def zrow(i, _):
        rows[0, i, :] = jnp.zeros((D,), _f32)
        return 0
    lax.fori_loop(0, GROUP_ROWS, zrow, 0)
    r0 = sid * RPT
    for q in range(RPT // GROUP_ROWS):
        pltpu.sync_copy(rows.at[0], agg_s.at[pl.ds(r0 + q * GROUP_ROWS, GROUP_ROWS)])
    rem = RPT % GROUP_ROWS
    if rem:
        pltpu.sync_copy(rows.at[0, pl.ds(0, rem)],
                        agg_s.at[pl.ds(r0 + RPT - rem, rem)])
    plsc.subcore_barrier()

    g0, ng = _worker_groups(wid)

    def fire_idx(g):
        base = (g0 + g) * KG
        pltpu.async_copy(src_hbm.at[pl.ds(base, KG)], sidx.at[g % 2], semi)
        pltpu.async_copy(dst_hbm.at[pl.ds(base, KG)], didx.at[g % 3], semi)

    def drain_idx():
        for _ in range(2):
            pltpu.make_async_copy(
                src_hbm.at[pl.ds(0, KG)], sidx.at[0], semi).wait()

    def drain_scatter():
        for _ in range(KG):
            pltpu.make_async_copy(
                y_hbm.at[pl.ds(0, LANES)], rows.at[0, pl.ds(0, LANES)],
                sems).wait()

    fire_idx(0)

    def group(g, _):
        br = g % 2
        bd = g % 3

        @pl.when(g >= 2)
        def _():
            drain_scatter()

        drain_idx()

        @pl.when(g + 1 < ng)
        def _():
            fire_idx(g + 1)

        gd = []
        for j in range(KG):
            gd.append(pltpu.async_copy(
                y_hbm.at[sidx.at[br, j]],
                rows.at[br, pl.ds(j * LANES, LANES)], semg))
        for d in gd:
            d.wait()
        for j in range(KG):
            pltpu.async_copy(
                rows.at[br, pl.ds(j * LANES, LANES)], agg_s.at[didx.at[bd, j]],
                sems, add=True)
        return 0

    lax.fori_loop(0, ng, group, 0)
    drain_scatter()
    drain_scatter()
    plsc.subcore_barrier()

    off = cid * NP + sid * RPT
    pltpu.sync_copy(agg_s.at[pl.ds(sid * RPT, RPT)], agg_hbm.at[pl.ds(off, RPT)])


@functools.lru_cache(maxsize=None)
def _sc_kernels():
    mesh = plsc.VectorSubcoreMesh(
        core_axis_name="c", subcore_axis_name="s",
        num_cores=NC, num_subcores=NS)
    params = pltpu.CompilerParams(use_tc_tiling_on_sc=False)
    sc_degrees = pl.kernel(
        _sc_degrees_body,
        out_type=(
            jax.ShapeDtypeStruct((NC * NP,), _f32),
            jax.ShapeDtypeStruct((NC * NP,), _f32),
        ),
        mesh=mesh,
        scratch_types=[
            pltpu.VMEM((3, KG, LANES), _i32),
            pltpu.VMEM((3, KG, LANES), _i32),
            pltpu.VMEM((LANES,), _f32),
            pltpu.VMEM((RPT,), _f32),
            pltpu.VMEM_SHARED((NP,), _f32),
            pltpu.VMEM_SHARED((NP,), _f32),
            pltpu.SemaphoreType.DMA,
            pltpu.SemaphoreType.DMA,
        ],
        compiler_params=params,
    )
    sc_message = pl.kernel(
        _sc_message_body,
        out_type=jax.ShapeDtypeStruct((NC * NP, D), _f32),
        mesh=mesh,
        scratch_types=[
            pltpu.VMEM((2, KG, LANES), _i32),
            pltpu.VMEM((3, KG, LANES), _i32),
            pltpu.VMEM((2, KG * LANES, D), _f32),
            pltpu.VMEM_SHARED((NP, D), _f32),
            pltpu.SemaphoreType.DMA,
            pltpu.SemaphoreType.DMA,
            pltpu.SemaphoreType.DMA,
        ],
        compiler_params=params,
    )
    return sc_degrees, sc_message


def _scale_packed(s8):
    tt = lax.broadcasted_iota(_i32, (D, LANES), 0)
    mm = lax.broadcasted_iota(_i32, (D, LANES), 1)
    sell = (mm // 8 == tt).astype(_f32)
    m2 = lax.broadcasted_iota(_i32, (LANES, LANES), 0)
    c2 = lax.broadcasted_iota(_i32, (LANES, LANES), 1)
    selr = (c2 // D == m2 % 8).astype(_f32)
    chunks = []
    for q in range(8):
        sq = s8[q:q + 1, :]
        chunks.append(jnp.dot(sell * sq, selr,
                              preferred_element_type=_f32))
    return jnp.concatenate(chunks, axis=0)


def _tc_prep_body(x_ref, do0_ref, do1_ref, y_ref):
    s_out = lax.rsqrt(do0_ref[0] + do1_ref[0] + 1.0)
    y_ref[...] = x_ref[...] * _scale_packed(s_out)


def _tc_combine1_body(p0_ref, p1_ref, yc_ref, x0_ref,
                      di0_ref, di1_ref, do0_ref, do1_ref, w_ref, y2_ref):
    s_in = lax.rsqrt(di0_ref[0] + di1_ref[0] + 1.0)
    s_out = lax.rsqrt(do0_ref[0] + do1_ref[0] + 1.0)
    agg = (p0_ref[...] + p1_ref[...] + yc_ref[...]) * _scale_packed(s_in)
    h = (1.0 - ALPHA) * agg + ALPHA * x0_ref[...]
    hw = jnp.dot(h, w_ref[...], preferred_element_type=_f32)
    x1 = jnp.maximum((1.0 - BETA1) * h + BETA1 * hw, 0.0)
    y2_ref[...] = x1 * _scale_packed(s_out)


def _tc_combine2_body(p0_ref, p1_ref, yc_ref, x0_ref,
                      di0_ref, di1_ref, w_ref,
                      d1w_ref, d1b_ref, d2wt_ref, d2b_ref, o_ref, acc_ref):
    i = pl.program_id(0)

    @pl.when(i == 0)
    def _():
        acc_ref[...] = jnp.zeros((1, LANES), _f32)

    s_in = lax.rsqrt(di0_ref[0] + di1_ref[0] + 1.0)
    agg = (p0_ref[...] + p1_ref[...] + yc_ref[...]) * _scale_packed(s_in)
    h = (1.0 - ALPHA) * agg + ALPHA * x0_ref[...]
    hw = jnp.dot(h, w_ref[...], preferred_element_type=_f32)
    x2 = jnp.maximum((1.0 - BETA2) * h + BETA2 * hw, 0.0)
    acc_ref[...] += jnp.sum(x2, axis=0, keepdims=True)

    @pl.when(i == G - 1)
    def _():
        c2 = lax.broadcasted_iota(_i32, (LANES, D), 0)
        f2 = lax.broadcasted_iota(_i32, (LANES, D), 1)
        fold = (c2 % D == f2).astype(_f32)
        pooled = jnp.dot(acc_ref[...], fold,
                         preferred_element_type=_f32) * (1.0 / N)
        hm = jnp.dot(pooled, d1w_ref[...], preferred_element_type=_f32)
        hm = jnp.maximum(hm + d1b_ref[...], 0.0)
        z = jnp.sum(hm * d2wt_ref[...], axis=1, keepdims=True) + d2b_ref[...]
        o_ref[...] = 1.0 / (1.0 + jnp.exp(-z))


_pk_spec = pl.BlockSpec((PB, LANES), lambda i: (i, 0))
_pk_spec_hi = pl.BlockSpec((PB, LANES), lambda i: (G + i, 0))
_vec_spec = pl.BlockSpec((1, 8, LANES), lambda i: (i, 0, 0))
_wb_spec = pl.BlockSpec((LANES, LANES), lambda i: (0, 0))


def _tc_prep(x_pk, dego0, dego1):
    return pl.pallas_call(
        _tc_prep_body,
        grid=(G,),
        in_specs=[_pk_spec, _vec_spec, _vec_spec],
        out_specs=_pk_spec,
        out_shape=jax.ShapeDtypeStruct((PR, LANES), _f32),
    )(x_pk, dego0, dego1)


def _tc_combine1(agg_pk, ycur, x0, degi0, degi1, dego0, dego1, wb):
    return pl.pallas_call(
        _tc_combine1_body,
        grid=(G,),
        in_specs=[_pk_spec, _pk_spec_hi, _pk_spec, _pk_spec,
                  _vec_spec, _vec_spec, _vec_spec, _vec_spec, _wb_spec],
        out_specs=_pk_spec,
        out_shape=jax.ShapeDtypeStruct((PR, LANES), _f32),
    )(agg_pk, agg_pk, ycur, x0, degi0, degi1, dego0, dego1, wb)


def _tc_combine2(agg_pk, ycur, x0, degi0, degi1, wb, d1w, d1b, d2wt, d2b):
    return pl.pallas_call(
        _tc_combine2_body,
        grid=(G,),
        in_specs=[_pk_spec, _pk_spec_hi, _pk_spec, _pk_spec,
                  _vec_spec, _vec_spec, _wb_spec,
                  pl.BlockSpec((D, PRED_H), lambda i: (0, 0)),
                  pl.BlockSpec((1, PRED_H), lambda i: (0, 0)),
                  pl.BlockSpec((1, PRED_H), lambda i: (0, 0)),
                  pl.BlockSpec((1, 1), lambda i: (0, 0))],
        out_specs=pl.BlockSpec((1, 1), lambda i: (0, 0)),
        out_shape=jax.ShapeDtypeStruct((1, 1), _f32),
        scratch_shapes=[pltpu.VMEM((1, LANES), _f32)],
    )(agg_pk, agg_pk, ycur, x0, degi0, degi1, wb, d1w, d1b, d2wt, d2b)


def kernel(x, edge_index, w1, w2, dec1_w, dec1_b, dec2_w, dec2_b):
    src_p = edge_index[0].reshape(EROWS, LANES)
    dst_p = edge_index[1].reshape(EROWS, LANES)
    x_pk = jnp.concatenate(
        [x, jnp.zeros((NP - N, D), _f32)], axis=0).reshape(PR, LANES)
    eye8 = jnp.eye(8, dtype=_f32)
    w1b = jnp.kron(eye8, w1)
    w2b = jnp.kron(eye8, w2)

    sc_degrees, sc_message = _sc_kernels()
    dego_f, degi_f = sc_degrees(src_p, dst_p)
    dego0 = dego_f[:NP].reshape(G, 8, LANES)
    dego1 = dego_f[NP:].reshape(G, 8, LANES)
    degi0 = degi_f[:NP].reshape(G, 8, LANES)
    degi1 = degi_f[NP:].reshape(G, 8, LANES)

    y1 = _tc_prep(x_pk, dego0, dego1)

    agg1 = sc_message(y1.reshape(NP, D), src_p, dst_p)
    agg1_pk = agg1.reshape(2 * PR, LANES)

    y2 = _tc_combine1(agg1_pk, y1, x_pk, degi0, degi1, dego0, dego1, w1b)

    agg2 = sc_message(y2.reshape(NP, D), src_p, dst_p)
    agg2_pk = agg2.reshape(2 * PR, LANES)

    o = _tc_combine2(agg2_pk, y2, x_pk, degi0, degi1, w2b,
                     dec1_w, dec1_b.reshape(1, PRED_H),
                     dec2_w.reshape(1, PRED_H), dec2_b.reshape(1, 1))
    return o


# speedup vs baseline: 85.6460x; 1.3150x over previous
import functools

import jax
import jax.numpy as jnp
import numpy as _np
from jax import lax
from jax.experimental import pallas as pl
from jax.experimental.pallas import tpu as pltpu
from jax.experimental.pallas import tpu_sc as plsc

N = 100000
E = 3200000
D = 16
PRED_H = 32
ALPHA = 0.5
BETA1 = float(_np.log(1.0 / 1.0 + 1.0))
BETA2 = float(_np.log(1.0 / 2.0 + 1.0))

NC = 2
NS = 16
NW = NC * NS
LANES = 128

NP = 100352
RPT = NP // NS
EROWS = E // LANES
KG = 4
GTOT = EROWS // KG
GBASE = GTOT // NW
GREM = GTOT % NW

PR = NP // 8
RB = 1024
PB = RB // 8
G = NP // RB

_f32 = jnp.float32
_i32 = jnp.int32


def _fill(ref, n, value):
    def body(i, _):
        ref[pl.ds(i * 16, 16)] = jnp.full((16,), value, _f32)
        return 0
    lax.fori_loop(0, n // 16, body, 0)


def _worker_groups(wid):
    extra = jnp.minimum(wid, GREM)
    return wid * GBASE + extra, GBASE + jnp.where(wid < GREM, 1, 0)


def _sc_degrees_body(src_hbm, dst_hbm, dego_hbm, degi_hbm,
                     sidx, didx, ones_v, zbuf, dego_s, degi_s, semi, semd):
    cid = lax.axis_index("c")
    sid = lax.axis_index("s")
    wid = sid * NC + cid

    _fill(ones_v, LANES, 1.0)
    _fill(zbuf, RPT, 0.0)
    pltpu.sync_copy(zbuf, dego_s.at[pl.ds(sid * RPT, RPT)])
    pltpu.sync_copy(zbuf, degi_s.at[pl.ds(sid * RPT, RPT)])
    plsc.subcore_barrier()

    g0, ng = _worker_groups(wid)

    def fire_idx(g):
        base = (g0 + g) * KG
        pltpu.async_copy(src_hbm.at[pl.ds(base, KG)], sidx.at[g % 3], semi)
        pltpu.async_copy(dst_hbm.at[pl.ds(base, KG)], didx.at[g % 3], semi)

    def drain_idx():
        for _ in range(2):
            pltpu.make_async_copy(
                src_hbm.at[pl.ds(0, KG)], sidx.at[0], semi).wait()

    def drain_scatter():
        for _ in range(2 * KG):
            pltpu.make_async_copy(
                src_hbm.at[pl.ds(0, 1)], sidx.at[0, pl.ds(0, 1)], semd).wait()

    fire_idx(0)

    def group(g, _):
        b = g % 3

        @pl.when(g >= 2)
        def _():
            drain_scatter()

        drain_idx()

        @pl.when(g + 1 < ng)
        def _():
            fire_idx(g + 1)

        for j in range(KG):
            pltpu.async_copy(ones_v, dego_s.at[sidx.at[b, j]], semd, add=True)
            pltpu.async_copy(ones_v, degi_s.at[didx.at[b, j]], semd, add=True)
        return 0

    lax.fori_loop(0, ng, group, 0)
    drain_scatter()
    drain_scatter()
    plsc.subcore_barrier()

    off = cid * NP + sid * RPT
    pltpu.sync_copy(dego_s.at[pl.ds(sid * RPT, RPT)], dego_hbm.at[pl.ds(off, RPT)])
    pltpu.sync_copy(degi_s.at[pl.ds(sid * RPT, RPT)], degi_hbm.at[pl.ds(off, RPT)])


def _sc_message_body(y_hbm, src_hbm, dst_hbm, agg_hbm,
                     sidx, didx, rows, agg_s, semi, semg, sems):
    cid = lax.axis_index("c")
    sid = lax.axis_index("s")
    wid = sid * NC + cid
    GROUP_ROWS = KG * LANES

    def zrow(i, _):
        rows[0, i, :] = jnp.zeros((D,), _f32)
        return 0
    lax.fori_loop(0, GROUP_ROWS, zrow, 0)
    r0 = sid * RPT
    for q in range(RPT // GROUP_ROWS):
        pltpu.sync_copy(rows.at[0], agg_s.at[pl.ds(r0 + q * GROUP_ROWS, GROUP_ROWS)])
    rem = RPT % GROUP_ROWS
    if rem:
        pltpu.sync_copy(rows.at[0, pl.ds(0, rem)],
                        agg_s.at[pl.ds(r0 + RPT - rem, rem)])
    plsc.subcore_barrier()

    g0, ng = _worker_groups(wid)

    def fire_idx(g):
        base = (g0 + g) * KG
        pltpu.async_copy(src_hbm.at[pl.ds(base, KG)], sidx.at[g % 4], semi)
        pltpu.async_copy(dst_hbm.at[pl.ds(base, KG)], didx.at[g % 4], semi)

    def drain_idx():
        for _ in range(2):
            pltpu.make_async_copy(
                src_hbm.at[pl.ds(0, KG)], sidx.at[0], semi).wait()

    def fire_gathers(g):
        for j in range(KG):
            pltpu.async_copy(
                y_hbm.at[sidx.at[g % 4, j]],
                rows.at[g % 3, pl.ds(j * LANES, LANES)], semg.at[g % 2])

    def drain_gathers(g):
        for _ in range(KG):
            pltpu.make_async_copy(
                y_hbm.at[pl.ds(0, LANES)], rows.at[0, pl.ds(0, LANES)],
                semg.at[g % 2]).wait()

    def drain_scatter():
        for _ in range(KG):
            pltpu.make_async_copy(
                y_hbm.at[pl.ds(0, LANES)], rows.at[0, pl.ds(0, LANES)],
                sems).wait()

    fire_idx(0)
    fire_idx(1)
    drain_idx()
    fire_gathers(0)

    def group(g, _):
        @pl.when(g >= 2)
        def _():
            drain_scatter()

        @pl.when(g + 1 < ng)
        def _():
            drain_idx()
            @pl.when(g + 2 < ng)
            def _():
                fire_idx(g + 2)
            fire_gathers(g + 1)

        drain_gathers(g)
        for j in range(KG):
            pltpu.async_copy(
                rows.at[g % 3, pl.ds(j * LANES, LANES)],
                agg_s.at[didx.at[g % 4, j]], sems, add=True)
        return 0

    lax.fori_loop(0, ng, group, 0)
    drain_scatter()
    drain_scatter()
    plsc.subcore_barrier()

    off = cid * NP + sid * RPT
    pltpu.sync_copy(agg_s.at[pl.ds(sid * RPT, RPT)], agg_hbm.at[pl.ds(off, RPT)])


@functools.lru_cache(maxsize=None)
def _sc_kernels():
    mesh = plsc.VectorSubcoreMesh(
        core_axis_name="c", subcore_axis_name="s",
        num_cores=NC, num_subcores=NS)
    params = pltpu.CompilerParams(use_tc_tiling_on_sc=False)
    sc_degrees = pl.kernel(
        _sc_degrees_body,
        out_type=(
            jax.ShapeDtypeStruct((NC * NP,), _f32),
            jax.ShapeDtypeStruct((NC * NP,), _f32),
        ),
        mesh=mesh,
        scratch_types=[
            pltpu.VMEM((3, KG, LANES), _i32),
            pltpu.VMEM((3, KG, LANES), _i32),
            pltpu.VMEM((LANES,), _f32),
            pltpu.VMEM((RPT,), _f32),
            pltpu.VMEM_SHARED((NP,), _f32),
            pltpu.VMEM_SHARED((NP,), _f32),
            pltpu.SemaphoreType.DMA,
            pltpu.SemaphoreType.DMA,
        ],
        compiler_params=params,
    )
    sc_message = pl.kernel(
        _sc_message_body,
        out_type=jax.ShapeDtypeStruct((NC * NP, D), _f32),
        mesh=mesh,
        scratch_types=[
            pltpu.VMEM((4, KG, LANES), _i32),
            pltpu.VMEM((4, KG, LANES), _i32),
            pltpu.VMEM((3, KG * LANES, D), _f32),
            pltpu.VMEM_SHARED((NP, D), _f32),
            pltpu.SemaphoreType.DMA,
            pltpu.SemaphoreType.DMA((2,)),
            pltpu.SemaphoreType.DMA,
        ],
        compiler_params=params,
    )
    return sc_degrees, sc_message


def _scale_packed(s8):
    tt = lax.broadcasted_iota(_i32, (D, LANES), 0)
    mm = lax.broadcasted_iota(_i32, (D, LANES), 1)
    sell = (mm // 8 == tt).astype(_f32)
    m2 = lax.broadcasted_iota(_i32, (LANES, LANES), 0)
    c2 = lax.broadcasted_iota(_i32, (LANES, LANES), 1)
    selr = (c2 // D == m2 % 8).astype(_f32)
    chunks = []
    for q in range(8):
        sq = s8[q:q + 1, :]
        chunks.append(jnp.dot(sell * sq, selr,
                              preferred_element_type=_f32))
    return jnp.concatenate(chunks, axis=0)


def _tc_prep_body(x_ref, do0_ref, do1_ref, y_ref):
    s_out = lax.rsqrt(do0_ref[0] + do1_ref[0] + 1.0)
    y_ref[...] = x_ref[...] * _scale_packed(s_out)


def _tc_combine1_body(p0_ref, p1_ref, yc_ref, x0_ref,
                      di0_ref, di1_ref, do0_ref, do1_ref, w_ref, y2_ref):
    s_in = lax.rsqrt(di0_ref[0] + di1_ref[0] + 1.0)
    s_out = lax.rsqrt(do0_ref[0] + do1_ref[0] + 1.0)
    agg = (p0_ref[...] + p1_ref[...] + yc_ref[...]) * _scale_packed(s_in)
    h = (1.0 - ALPHA) * agg + ALPHA * x0_ref[...]
    hw = jnp.dot(h, w_ref[...], preferred_element_type=_f32)
    x1 = jnp.maximum((1.0 - BETA1) * h + BETA1 * hw, 0.0)
    y2_ref[...] = x1 * _scale_packed(s_out)


def _tc_combine2_body(p0_ref, p1_ref, yc_ref, x0_ref,
                      di0_ref, di1_ref, w_ref,
                      d1w_ref, d1b_ref, d2wt_ref, d2b_ref, o_ref, acc_ref):
    i = pl.program_id(0)

    @pl.when(i == 0)
    def _():
        acc_ref[...] = jnp.zeros((1, LANES), _f32)

    s_in = lax.rsqrt(di0_ref[0] + di1_ref[0] + 1.0)
    agg = (p0_ref[...] + p1_ref[...] + yc_ref[...]) * _scale_packed(s_in)
    h = (1.0 - ALPHA) * agg + ALPHA * x0_ref[...]
    hw = jnp.dot(h, w_ref[...], preferred_element_type=_f32)
    x2 = jnp.maximum((1.0 - BETA2) * h + BETA2 * hw, 0.0)
    acc_ref[...] += jnp.sum(x2, axis=0, keepdims=True)

    @pl.when(i == G - 1)
    def _():
        c2 = lax.broadcasted_iota(_i32, (LANES, D), 0)
        f2 = lax.broadcasted_iota(_i32, (LANES, D), 1)
        fold = (c2 % D == f2).astype(_f32)
        pooled = jnp.dot(acc_ref[...], fold,
                         preferred_element_type=_f32) * (1.0 / N)
        hm = jnp.dot(pooled, d1w_ref[...], preferred_element_type=_f32)
        hm = jnp.maximum(hm + d1b_ref[...], 0.0)
        z = jnp.sum(hm * d2wt_ref[...], axis=1, keepdims=True) + d2b_ref[...]
        o_ref[...] = 1.0 / (1.0 + jnp.exp(-z))


_pk_spec = pl.BlockSpec((PB, LANES), lambda i: (i, 0))
_pk_spec_hi = pl.BlockSpec((PB, LANES), lambda i: (G + i, 0))
_vec_spec = pl.BlockSpec((1, 8, LANES), lambda i: (i, 0, 0))
_wb_spec = pl.BlockSpec((LANES, LANES), lambda i: (0, 0))


def _tc_prep(x_pk, dego0, dego1):
    return pl.pallas_call(
        _tc_prep_body,
        grid=(G,),
        in_specs=[_pk_spec, _vec_spec, _vec_spec],
        out_specs=_pk_spec,
        out_shape=jax.ShapeDtypeStruct((PR, LANES), _f32),
    )(x_pk, dego0, dego1)


def _tc_combine1(agg_pk, ycur, x0, degi0, degi1, dego0, dego1, wb):
    return pl.pallas_call(
        _tc_combine1_body,
        grid=(G,),
        in_specs=[_pk_spec, _pk_spec_hi, _pk_spec, _pk_spec,
                  _vec_spec, _vec_spec, _vec_spec, _vec_spec, _wb_spec],
        out_specs=_pk_spec,
        out_shape=jax.ShapeDtypeStruct((PR, LANES), _f32),
    )(agg_pk, agg_pk, ycur, x0, degi0, degi1, dego0, dego1, wb)


def _tc_combine2(agg_pk, ycur, x0, degi0, degi1, wb, d1w, d1b, d2wt, d2b):
    return pl.pallas_call(
        _tc_combine2_body,
        grid=(G,),
        in_specs=[_pk_spec, _pk_spec_hi, _pk_spec, _pk_spec,
                  _vec_spec, _vec_spec, _wb_spec,
                  pl.BlockSpec((D, PRED_H), lambda i: (0, 0)),
                  pl.BlockSpec((1, PRED_H), lambda i: (0, 0)),
                  pl.BlockSpec((1, PRED_H), lambda i: (0, 0)),
                  pl.BlockSpec((1, 1), lambda i: (0, 0))],
        out_specs=pl.BlockSpec((1, 1), lambda i: (0, 0)),
        out_shape=jax.ShapeDtypeStruct((1, 1), _f32),
        scratch_shapes=[pltpu.VMEM((1, LANES), _f32)],
    )(agg_pk, agg_pk, ycur, x0, degi0, degi1, wb, d1w, d1b, d2wt, d2b)


def kernel(x, edge_index, w1, w2, dec1_w, dec1_b, dec2_w, dec2_b):
    src_p = edge_index[0].reshape(EROWS, LANES)
    dst_p = edge_index[1].reshape(EROWS, LANES)
    x_pk = jnp.concatenate(
        [x, jnp.zeros((NP - N, D), _f32)], axis=0).reshape(PR, LANES)
    eye8 = jnp.eye(8, dtype=_f32)
    w1b = jnp.kron(eye8, w1)
    w2b = jnp.kron(eye8, w2)

    sc_degrees, sc_message = _sc_kernels()
    dego_f, degi_f = sc_degrees(src_p, dst_p)
    dego0 = dego_f[:NP].reshape(G, 8, LANES)
    dego1 = dego_f[NP:].reshape(G, 8, LANES)
    degi0 = degi_f[:NP].reshape(G, 8, LANES)
    degi1 = degi_f[NP:].reshape(G, 8, LANES)

    y1 = _tc_prep(x_pk, dego0, dego1)

    agg1 = sc_message(y1.reshape(NP, D), src_p, dst_p)
    agg1_pk = agg1.reshape(2 * PR, LANES)

    y2 = _tc_combine1(agg1_pk, y1, x_pk, degi0, degi1, dego0, dego1, w1b)

    agg2 = sc_message(y2.reshape(NP, D), src_p, dst_p)
    agg2_pk = agg2.reshape(2 * PR, LANES)

    o = _tc_combine2(agg2_pk, y2, x_pk, degi0, degi1, w2b,
                     dec1_w, dec1_b.reshape(1, PRED_H),
                     dec2_w.reshape(1, PRED_H), dec2_b.reshape(1, 1))
    return o


# speedup vs baseline: 87.0293x; 1.0162x over previous
import functools

import jax
import jax.numpy as jnp
import numpy as _np
from jax import lax
from jax.experimental import pallas as pl
from jax.experimental.pallas import tpu as pltpu
from jax.experimental.pallas import tpu_sc as plsc

N = 100000
E = 3200000
D = 16
PRED_H = 32
ALPHA = 0.5
BETA1 = float(_np.log(1.0 / 1.0 + 1.0))
BETA2 = float(_np.log(1.0 / 2.0 + 1.0))

NC = 2
NS = 16
NW = NC * NS
LANES = 128

NP = 100352
RPT = NP // NS
EROWS = E // LANES
KG = 4
GTOT = EROWS // KG
GBASE = GTOT // NW
GREM = GTOT % NW

PR = NP // 8
PRX = N * D // LANES
RB = 1024
PB = RB // 8
G = NP // RB

_f32 = jnp.float32
_i32 = jnp.int32


def _fill(ref, n, value):
    def body(i, _):
        ref[pl.ds(i * 16, 16)] = jnp.full((16,), value, _f32)
        return 0
    lax.fori_loop(0, n // 16, body, 0)


def _worker_groups(wid):
    extra = jnp.minimum(wid, GREM)
    return wid * GBASE + extra, GBASE + jnp.where(wid < GREM, 1, 0)


def _sc_degrees_body(src_hbm, dst_hbm, dego_hbm, degi_hbm,
                     sidx, didx, ones_v, zbuf, dego_s, degi_s, semi, semd):
    cid = lax.axis_index("c")
    sid = lax.axis_index("s")
    wid = sid * NC + cid

    _fill(ones_v, LANES, 1.0)
    _fill(zbuf, RPT, 0.0)
    pltpu.sync_copy(zbuf, dego_s.at[pl.ds(sid * RPT, RPT)])
    pltpu.sync_copy(zbuf, degi_s.at[pl.ds(sid * RPT, RPT)])
    plsc.subcore_barrier()

    g0, ng = _worker_groups(wid)

    def fire_idx(g):
        base = (g0 + g) * KG
        pltpu.async_copy(src_hbm.at[pl.ds(base, KG)], sidx.at[g % 3], semi)
        pltpu.async_copy(dst_hbm.at[pl.ds(base, KG)], didx.at[g % 3], semi)

    def drain_idx():
        for _ in range(2):
            pltpu.make_async_copy(
                src_hbm.at[pl.ds(0, KG)], sidx.at[0], semi).wait()

    def drain_scatter():
        for _ in range(2 * KG):
            pltpu.make_async_copy(
                src_hbm.at[pl.ds(0, 1)], sidx.at[0, pl.ds(0, 1)], semd).wait()

    fire_idx(0)

    def group(g, _):
        b = g % 3

        @pl.when(g >= 2)
        def _():
            drain_scatter()

        drain_idx()

        @pl.when(g + 1 < ng)
        def _():
            fire_idx(g + 1)

        for j in range(KG):
            pltpu.async_copy(ones_v, dego_s.at[sidx.at[b, j]], semd, add=True)
            pltpu.async_copy(ones_v, degi_s.at[didx.at[b, j]], semd, add=True)
        return 0

    lax.fori_loop(0, ng, group, 0)
    drain_scatter()
    drain_scatter()
    plsc.subcore_barrier()

    off = cid * NP + sid * RPT
    pltpu.sync_copy(dego_s.at[pl.ds(sid * RPT, RPT)], dego_hbm.at[pl.ds(off, RPT)])
    pltpu.sync_copy(degi_s.at[pl.ds(sid * RPT, RPT)], degi_hbm.at[pl.ds(off, RPT)])


def _sc_message_body(y_hbm, src_hbm, dst_hbm, agg_hbm,
                     sidx, didx, rows, agg_s, semi, semg, sems):
    cid = lax.axis_index("c")
    sid = lax.axis_index("s")
    wid = sid * NC + cid
    GROUP_ROWS = KG * LANES

    def zrow(i, _):
        rows[0, i, :] = jnp.zeros((D,), _f32)
        return 0
    lax.fori_loop(0, GROUP_ROWS, zrow, 0)
    r0 = sid * RPT
    for q in range(RPT // GROUP_ROWS):
        pltpu.sync_copy(rows.at[0], agg_s.at[pl.ds(r0 + q * GROUP_ROWS, GROUP_ROWS)])
    rem = RPT % GROUP_ROWS
    if rem:
        pltpu.sync_copy(rows.at[0, pl.ds(0, rem)],
                        agg_s.at[pl.ds(r0 + RPT - rem, rem)])
    plsc.subcore_barrier()

    g0, ng = _worker_groups(wid)

    def fire_idx(g):
        base = (g0 + g) * KG
        pltpu.async_copy(src_hbm.at[pl.ds(base, KG)], sidx.at[g % 4], semi)
        pltpu.async_copy(dst_hbm.at[pl.ds(base, KG)], didx.at[g % 4], semi)

    def drain_idx():
        for _ in range(2):
            pltpu.make_async_copy(
                src_hbm.at[pl.ds(0, KG)], sidx.at[0], semi).wait()

    def fire_gathers(g):
        for j in range(KG):
            pltpu.async_copy(
                y_hbm.at[sidx.at[g % 4, j]],
                rows.at[g % 3, pl.ds(j * LANES, LANES)], semg.at[g % 2])

    def drain_gathers(g):
        for _ in range(KG):
            pltpu.make_async_copy(
                y_hbm.at[pl.ds(0, LANES)], rows.at[0, pl.ds(0, LANES)],
                semg.at[g % 2]).wait()

    def drain_scatter():
        for _ in range(KG):
            pltpu.make_async_copy(
                y_hbm.at[pl.ds(0, LANES)], rows.at[0, pl.ds(0, LANES)],
                sems).wait()

    fire_idx(0)
    fire_idx(1)
    drain_idx()
    fire_gathers(0)

    def group(g, _):
        @pl.when(g >= 2)
        def _():
            drain_scatter()

        @pl.when(g + 1 < ng)
        def _():
            drain_idx()
            @pl.when(g + 2 < ng)
            def _():
                fire_idx(g + 2)
            fire_gathers(g + 1)

        drain_gathers(g)
        for j in range(KG):
            pltpu.async_copy(
                rows.at[g % 3, pl.ds(j * LANES, LANES)],
                agg_s.at[didx.at[g % 4, j]], sems, add=True)
        return 0

    lax.fori_loop(0, ng, group, 0)
    drain_scatter()
    drain_scatter()
    plsc.subcore_barrier()

    off = cid * NP + sid * RPT
    pltpu.sync_copy(agg_s.at[pl.ds(sid * RPT, RPT)], agg_hbm.at[pl.ds(off, RPT)])


@functools.lru_cache(maxsize=None)
def _sc_kernels():
    mesh = plsc.VectorSubcoreMesh(
        core_axis_name="c", subcore_axis_name="s",
        num_cores=NC, num_subcores=NS)
    params = pltpu.CompilerParams(use_tc_tiling_on_sc=False)
    sc_degrees = pl.kernel(
        _sc_degrees_body,
        out_type=(
            jax.ShapeDtypeStruct((NC * NP,), _f32),
            jax.ShapeDtypeStruct((NC * NP,), _f32),
        ),
        mesh=mesh,
        scratch_types=[
            pltpu.VMEM((3, KG, LANES), _i32),
            pltpu.VMEM((3, KG, LANES), _i32),
            pltpu.VMEM((LANES,), _f32),
            pltpu.VMEM((RPT,), _f32),
            pltpu.VMEM_SHARED((NP,), _f32),
            pltpu.VMEM_SHARED((NP,), _f32),
            pltpu.SemaphoreType.DMA,
            pltpu.SemaphoreType.DMA,
        ],
        compiler_params=params,
    )
    sc_message = pl.kernel(
        _sc_message_body,
        out_type=jax.ShapeDtypeStruct((NC * NP, D), _f32),
        mesh=mesh,
        scratch_types=[
            pltpu.VMEM((4, KG, LANES), _i32),
            pltpu.VMEM((4, KG, LANES), _i32),
            pltpu.VMEM((3, KG * LANES, D), _f32),
            pltpu.VMEM_SHARED((NP, D), _f32),
            pltpu.SemaphoreType.DMA,
            pltpu.SemaphoreType.DMA((2,)),
            pltpu.SemaphoreType.DMA,
        ],
        compiler_params=params,
    )
    return sc_degrees, sc_message


def _scale_packed(s8):
    rr = lax.broadcasted_iota(_i32, (LANES, 8), 0)
    qq = lax.broadcasted_iota(_i32, (LANES, 8), 1)
    fold8 = (rr // D == qq).astype(_f32)
    s_exp = jnp.dot(fold8, s8, preferred_element_type=_f32)
    tt = lax.broadcasted_iota(_i32, (LANES, LANES), 0)
    mm = lax.broadcasted_iota(_i32, (LANES, LANES), 1)
    sell = (mm // 8 == tt % D).astype(_f32)
    m2 = lax.broadcasted_iota(_i32, (LANES, LANES), 0)
    c2 = lax.broadcasted_iota(_i32, (LANES, LANES), 1)
    selr = (c2 // D == m2 % 8).astype(_f32)
    return jnp.dot(sell * s_exp, selr, preferred_element_type=_f32)


def _tc_prep_body(x_ref, do0_ref, do1_ref, y_ref):
    i = pl.program_id(0)
    s_out = lax.rsqrt(do0_ref[0] + do1_ref[0] + 1.0)
    y_ref[...] = x_ref[...] * _scale_packed(s_out)

    @pl.when(i == G - 1)
    def _():
        y_ref[pl.ds(PRX - (G - 1) * PB, PB * G - PRX), :] = jnp.zeros(
            (PB * G - PRX, LANES), _f32)


def _tc_combine1_body(p0_ref, p1_ref, yc_ref,
                      di0_ref, di1_ref, do0_ref, do1_ref, w_ref, y2_ref):
    do8 = do0_ref[0] + do1_ref[0] + 1.0
    s_in = lax.rsqrt(di0_ref[0] + di1_ref[0] + 1.0)
    yc = yc_ref[...]
    agg = (p0_ref[...] + p1_ref[...] + yc) * _scale_packed(s_in)
    h = (1.0 - ALPHA) * agg + ALPHA * (yc * _scale_packed(lax.sqrt(do8)))
    hw = jnp.dot(h, w_ref[...], preferred_element_type=_f32)
    x1 = jnp.maximum((1.0 - BETA1) * h + BETA1 * hw, 0.0)
    y2_ref[...] = x1 * _scale_packed(lax.rsqrt(do8))


def _tc_combine2_body(p0_ref, p1_ref, yc_ref, y1_ref,
                      di0_ref, di1_ref, do0_ref, do1_ref, w_ref,
                      d1w_ref, d1b_ref, d2wt_ref, d2b_ref, o_ref, acc_ref):
    i = pl.program_id(0)

    @pl.when(i == 0)
    def _():
        acc_ref[...] = jnp.zeros((1, LANES), _f32)

    do8 = do0_ref[0] + do1_ref[0] + 1.0
    s_in = lax.rsqrt(di0_ref[0] + di1_ref[0] + 1.0)
    yc = yc_ref[...]
    agg = (p0_ref[...] + p1_ref[...] + yc) * _scale_packed(s_in)
    h = (1.0 - ALPHA) * agg + ALPHA * (y1_ref[...] * _scale_packed(lax.sqrt(do8)))
    hw = jnp.dot(h, w_ref[...], preferred_element_type=_f32)
    x2 = jnp.maximum((1.0 - BETA2) * h + BETA2 * hw, 0.0)
    acc_ref[...] += jnp.sum(x2, axis=0, keepdims=True)

    @pl.when(i == G - 1)
    def _():
        c2 = lax.broadcasted_iota(_i32, (LANES, D), 0)
        f2 = lax.broadcasted_iota(_i32, (LANES, D), 1)
        fold = (c2 % D == f2).astype(_f32)
        pooled = jnp.dot(acc_ref[...], fold,
                         preferred_element_type=_f32) * (1.0 / N)
        hm = jnp.dot(pooled, d1w_ref[...], preferred_element_type=_f32)
        hm = jnp.maximum(hm + d1b_ref[...], 0.0)
        z = jnp.sum(hm * d2wt_ref[...], axis=1, keepdims=True) + d2b_ref[...]
        o_ref[...] = 1.0 / (1.0 + jnp.exp(-z))


_pk_spec = pl.BlockSpec((PB, LANES), lambda i: (i, 0))
_pk_spec_hi = pl.BlockSpec((PB, LANES), lambda i: (G + i, 0))
_vec_spec = pl.BlockSpec((1, 8, LANES), lambda i: (i, 0, 0))
_wb_spec = pl.BlockSpec((LANES, LANES), lambda i: (0, 0))


def _tc_prep(x_v, dego0, dego1):
    return pl.pallas_call(
        _tc_prep_body,
        grid=(G,),
        in_specs=[_pk_spec, _vec_spec, _vec_spec],
        out_specs=_pk_spec,
        out_shape=jax.ShapeDtypeStruct((PR, LANES), _f32),
    )(x_v, dego0, dego1)


def _tc_combine1(agg_pk, ycur, degi0, degi1, dego0, dego1, wb):
    return pl.pallas_call(
        _tc_combine1_body,
        grid=(G,),
        in_specs=[_pk_spec, _pk_spec_hi, _pk_spec,
                  _vec_spec, _vec_spec, _vec_spec, _vec_spec, _wb_spec],
        out_specs=_pk_spec,
        out_shape=jax.ShapeDtypeStruct((PR, LANES), _f32),
    )(agg_pk, agg_pk, ycur, degi0, degi1, dego0, dego1, wb)


def _tc_combine2(agg_pk, ycur, y1, degi0, degi1, dego0, dego1, wb,
                 d1w, d1b, d2wt, d2b):
    return pl.pallas_call(
        _tc_combine2_body,
        grid=(G,),
        in_specs=[_pk_spec, _pk_spec_hi, _pk_spec, _pk_spec,
                  _vec_spec, _vec_spec, _vec_spec, _vec_spec, _wb_spec,
                  pl.BlockSpec((D, PRED_H), lambda i: (0, 0)),
                  pl.BlockSpec((1, PRED_H), lambda i: (0, 0)),
                  pl.BlockSpec((1, PRED_H), lambda i: (0, 0)),
                  pl.BlockSpec((1, 1), lambda i: (0, 0))],
        out_specs=pl.BlockSpec((1, 1), lambda i: (0, 0)),
        out_shape=jax.ShapeDtypeStruct((1, 1), _f32),
        scratch_shapes=[pltpu.VMEM((1, LANES), _f32)],
    )(agg_pk, agg_pk, ycur, y1, degi0, degi1, dego0, dego1, wb,
      d1w, d1b, d2wt, d2b)


def kernel(x, edge_index, w1, w2, dec1_w, dec1_b, dec2_w, dec2_b):
    src_p = edge_index[0].reshape(EROWS, LANES)
    dst_p = edge_index[1].reshape(EROWS, LANES)
    x_v = x.reshape(PRX, LANES)
    eye8 = jnp.eye(8, dtype=_f32)
    w1b = jnp.kron(eye8, w1)
    w2b = jnp.kron(eye8, w2)

    sc_degrees, sc_message = _sc_kernels()
    dego_f, degi_f = sc_degrees(src_p, dst_p)
    dego0 = dego_f[:NP].reshape(G, 8, LANES)
    dego1 = dego_f[NP:].reshape(G, 8, LANES)
    degi0 = degi_f[:NP].reshape(G, 8, LANES)
    degi1 = degi_f[NP:].reshape(G, 8, LANES)

    y1 = _tc_prep(x_v, dego0, dego1)

    agg1 = sc_message(y1.reshape(NP, D), src_p, dst_p)
    agg1_pk = agg1.reshape(2 * PR, LANES)

    y2 = _tc_combine1(agg1_pk, y1, degi0, degi1, dego0, dego1, w1b)

    agg2 = sc_message(y2.reshape(NP, D), src_p, dst_p)
    agg2_pk = agg2.reshape(2 * PR, LANES)

    o = _tc_combine2(agg2_pk, y2, y1, degi0, degi1, dego0, dego1, w2b,
                     dec1_w, dec1_b.reshape(1, PRED_H),
                     dec2_w.reshape(1, PRED_H), dec2_b.reshape(1, 1))
    return o


# speedup vs baseline: 99.8182x; 1.1469x over previous
import functools

import jax
import jax.numpy as jnp
import numpy as _np
from jax import lax
from jax.experimental import pallas as pl
from jax.experimental.pallas import tpu as pltpu
from jax.experimental.pallas import tpu_sc as plsc

N = 100000
E = 3200000
D = 16
PRED_H = 32
ALPHA = 0.5
BETA1 = float(_np.log(1.0 / 1.0 + 1.0))
BETA2 = float(_np.log(1.0 / 2.0 + 1.0))

NC = 2
NS = 16
NW = NC * NS
LANES = 128

NP = 100352
RPT = NP // NS
EROWS = E // LANES
KG = 4
GTOT = EROWS // KG
GBASE = GTOT // NW
GREM = GTOT % NW

PR = NP // 8
PRX = N * D // LANES
RB = 2048
PB = RB // 8
QQ = RB // LANES
G = NP // RB

_f32 = jnp.float32
_i32 = jnp.int32


def _fill(ref, n, value):
    def body(i, _):
        ref[pl.ds(i * 16, 16)] = jnp.full((16,), value, _f32)
        return 0
    lax.fori_loop(0, n // 16, body, 0)


def _worker_groups(wid):
    extra = jnp.minimum(wid, GREM)
    return wid * GBASE + extra, GBASE + jnp.where(wid < GREM, 1, 0)


def _sc_degrees_body(src_hbm, dst_hbm, dego_hbm, degi_hbm,
                     sidx, didx, ones_v, zbuf, dego_s, degi_s, semi, semd):
    cid = lax.axis_index("c")
    sid = lax.axis_index("s")
    wid = sid * NC + cid

    _fill(ones_v, LANES, 1.0)
    _fill(zbuf, RPT, 0.0)
    pltpu.sync_copy(zbuf, dego_s.at[pl.ds(sid * RPT, RPT)])
    pltpu.sync_copy(zbuf, degi_s.at[pl.ds(sid * RPT, RPT)])
    plsc.subcore_barrier()

    g0, ng = _worker_groups(wid)

    def fire_idx(g):
        base = (g0 + g) * KG
        pltpu.async_copy(src_hbm.at[pl.ds(base, KG)], sidx.at[g % 3], semi)
        pltpu.async_copy(dst_hbm.at[pl.ds(base, KG)], didx.at[g % 3], semi)

    def drain_idx():
        for _ in range(2):
            pltpu.make_async_copy(
                src_hbm.at[pl.ds(0, KG)], sidx.at[0], semi).wait()

    def drain_scatter():
        for _ in range(2 * KG):
            pltpu.make_async_copy(
                src_hbm.at[pl.ds(0, 1)], sidx.at[0, pl.ds(0, 1)], semd).wait()

    fire_idx(0)

    def group(g, _):
        b = g % 3

        @pl.when(g >= 2)
        def _():
            drain_scatter()

        drain_idx()

        @pl.when(g + 1 < ng)
        def _():
            fire_idx(g + 1)

        for j in range(KG):
            pltpu.async_copy(ones_v, dego_s.at[sidx.at[b, j]], semd, add=True)
            pltpu.async_copy(ones_v, degi_s.at[didx.at[b, j]], semd, add=True)
        return 0

    lax.fori_loop(0, ng, group, 0)
    drain_scatter()
    drain_scatter()
    plsc.subcore_barrier()

    off = cid * NP + sid * RPT
    pltpu.sync_copy(dego_s.at[pl.ds(sid * RPT, RPT)], dego_hbm.at[pl.ds(off, RPT)])
    pltpu.sync_copy(degi_s.at[pl.ds(sid * RPT, RPT)], degi_hbm.at[pl.ds(off, RPT)])


def _sc_message_body(y_hbm, src_hbm, dst_hbm, agg_hbm,
                     sidx, didx, rows, agg_s, semi, semg, sems):
    cid = lax.axis_index("c")
    sid = lax.axis_index("s")
    wid = sid * NC + cid
    GROUP_ROWS = KG * LANES

    def zrow(i, _):
        rows[0, i, :] = jnp.zeros((D,), _f32)
        return 0
    lax.fori_loop(0, GROUP_ROWS, zrow, 0)
    r0 = sid * RPT
    for q in range(RPT // GROUP_ROWS):
        pltpu.sync_copy(rows.at[0], agg_s.at[pl.ds(r0 + q * GROUP_ROWS, GROUP_ROWS)])
    rem = RPT % GROUP_ROWS
    if rem:
        pltpu.sync_copy(rows.at[0, pl.ds(0, rem)],
                        agg_s.at[pl.ds(r0 + RPT - rem, rem)])
    plsc.subcore_barrier()

    g0, ng = _worker_groups(wid)

    def fire_idx(g):
        base = (g0 + g) * KG
        pltpu.async_copy(src_hbm.at[pl.ds(base, KG)], sidx.at[g % 4], semi)
        pltpu.async_copy(dst_hbm.at[pl.ds(base, KG)], didx.at[g % 4], semi)

    def drain_idx():
        for _ in range(2):
            pltpu.make_async_copy(
                src_hbm.at[pl.ds(0, KG)], sidx.at[0], semi).wait()

    def fire_gathers(g):
        for j in range(KG):
            pltpu.async_copy(
                y_hbm.at[sidx.at[g % 4, j]],
                rows.at[g % 3, pl.ds(j * LANES, LANES)], semg.at[g % 2])

    def drain_gathers(g):
        for _ in range(KG):
            pltpu.make_async_copy(
                y_hbm.at[pl.ds(0, LANES)], rows.at[0, pl.ds(0, LANES)],
                semg.at[g % 2]).wait()

    def drain_scatter():
        for _ in range(KG):
            pltpu.make_async_copy(
                y_hbm.at[pl.ds(0, LANES)], rows.at[0, pl.ds(0, LANES)],
                sems).wait()

    fire_idx(0)
    fire_idx(1)
    drain_idx()
    fire_gathers(0)

    def group(g, _):
        @pl.when(g >= 2)
        def _():
            drain_scatter()

        @pl.when(g + 1 < ng)
        def _():
            drain_idx()
            @pl.when(g + 2 < ng)
            def _():
                fire_idx(g + 2)
            fire_gathers(g + 1)

        drain_gathers(g)
        for j in range(KG):
            pltpu.async_copy(
                rows.at[g % 3, pl.ds(j * LANES, LANES)],
                agg_s.at[didx.at[g % 4, j]], sems, add=True)
        return 0

    lax.fori_loop(0, ng, group, 0)
    drain_scatter()
    drain_scatter()
    plsc.subcore_barrier()

    off = cid * NP + sid * RPT
    pltpu.sync_copy(agg_s.at[pl.ds(sid * RPT, RPT)], agg_hbm.at[pl.ds(off, RPT)])


@functools.lru_cache(maxsize=None)
def _sc_kernels():
    mesh = plsc.VectorSubcoreMesh(
        core_axis_name="c", subcore_axis_name="s",
        num_cores=NC, num_subcores=NS)
    params = pltpu.CompilerParams(use_tc_tiling_on_sc=False)
    sc_degrees = pl.kernel(
        _sc_degrees_body,
        out_type=(
            jax.ShapeDtypeStruct((NC * NP,), _f32),
            jax.ShapeDtypeStruct((NC * NP,), _f32),
        ),
        mesh=mesh,
        scratch_types=[
            pltpu.VMEM((3, KG, LANES), _i32),
            pltpu.VMEM((3, KG, LANES), _i32),
            pltpu.VMEM((LANES,), _f32),
            pltpu.VMEM((RPT,), _f32),
            pltpu.VMEM_SHARED((NP,), _f32),
            pltpu.VMEM_SHARED((NP,), _f32),
            pltpu.SemaphoreType.DMA,
            pltpu.SemaphoreType.DMA,
        ],
        compiler_params=params,
    )
    sc_message = pl.kernel(
        _sc_message_body,
        out_type=jax.ShapeDtypeStruct((NC * NP, D), _f32),
        mesh=mesh,
        scratch_types=[
            pltpu.VMEM((4, KG, LANES), _i32),
            pltpu.VMEM((4, KG, LANES), _i32),
            pltpu.VMEM((3, KG * LANES, D), _f32),
            pltpu.VMEM_SHARED((NP, D), _f32),
            pltpu.SemaphoreType.DMA,
            pltpu.SemaphoreType.DMA((2,)),
            pltpu.SemaphoreType.DMA,
        ],
        compiler_params=params,
    )
    return sc_degrees, sc_message


def _scale_packed(s8):
    rr = lax.broadcasted_iota(_i32, (PB, QQ), 0)
    qq = lax.broadcasted_iota(_i32, (PB, QQ), 1)
    fold8 = (rr // D == qq).astype(_f32)
    s_exp = jnp.dot(fold8, s8, preferred_element_type=_f32)
    tt = lax.broadcasted_iota(_i32, (PB, LANES), 0)
    mm = lax.broadcasted_iota(_i32, (PB, LANES), 1)
    sell = (mm // 8 == tt % D).astype(_f32)
    m2 = lax.broadcasted_iota(_i32, (LANES, LANES), 0)
    c2 = lax.broadcasted_iota(_i32, (LANES, LANES), 1)
    selr = (c2 // D == m2 % 8).astype(_f32)
    return jnp.dot(sell * s_exp, selr, preferred_element_type=_f32)


def _tc_prep_body(x_ref, do0_ref, do1_ref, y_ref):
    i = pl.program_id(0)
    s_out = lax.rsqrt(do0_ref[0] + do1_ref[0] + 1.0)
    y_ref[...] = x_ref[...] * _scale_packed(s_out)

    @pl.when(i == G - 1)
    def _():
        y_ref[pl.ds(PRX - (G - 1) * PB, PB * G - PRX), :] = jnp.zeros(
            (PB * G - PRX, LANES), _f32)


def _tc_combine1_body(p0_ref, p1_ref, yc_ref,
                      di0_ref, di1_ref, do0_ref, do1_ref, w_ref, y2_ref):
    do8 = do0_ref[0] + do1_ref[0] + 1.0
    s_in = lax.rsqrt(di0_ref[0] + di1_ref[0] + 1.0)
    yc = yc_ref[...]
    agg = (p0_ref[...] + p1_ref[...] + yc) * _scale_packed(s_in)
    h = (1.0 - ALPHA) * agg + ALPHA * (yc * _scale_packed(lax.sqrt(do8)))
    hw = jnp.dot(h, w_ref[...], preferred_element_type=_f32)
    x1 = jnp.maximum((1.0 - BETA1) * h + BETA1 * hw, 0.0)
    y2_ref[...] = x1 * _scale_packed(lax.rsqrt(do8))


def _tc_combine2_body(p0_ref, p1_ref, yc_ref, y1_ref,
                      di0_ref, di1_ref, do0_ref, do1_ref, w_ref,
                      d1w_ref, d1b_ref, d2wt_ref, d2b_ref, o_ref, acc_ref):
    i = pl.program_id(0)

    @pl.when(i == 0)
    def _():
        acc_ref[...] = jnp.zeros((1, LANES), _f32)

    do8 = do0_ref[0] + do1_ref[0] + 1.0
    s_in = lax.rsqrt(di0_ref[0] + di1_ref[0] + 1.0)
    yc = yc_ref[...]
    agg = (p0_ref[...] + p1_ref[...] + yc) * _scale_packed(s_in)
    h = (1.0 - ALPHA) * agg + ALPHA * (y1_ref[...] * _scale_packed(lax.sqrt(do8)))
    hw = jnp.dot(h, w_ref[...], preferred_element_type=_f32)
    x2 = jnp.maximum((1.0 - BETA2) * h + BETA2 * hw, 0.0)
    acc_ref[...] += jnp.sum(x2, axis=0, keepdims=True)

    @pl.when(i == G - 1)
    def _():
        c2 = lax.broadcasted_iota(_i32, (LANES, D), 0)
        f2 = lax.broadcasted_iota(_i32, (LANES, D), 1)
        fold = (c2 % D == f2).astype(_f32)
        pooled = jnp.dot(acc_ref[...], fold,
                         preferred_element_type=_f32) * (1.0 / N)
        hm = jnp.dot(pooled, d1w_ref[...], preferred_element_type=_f32)
        hm = jnp.maximum(hm + d1b_ref[...], 0.0)
        z = jnp.sum(hm * d2wt_ref[...], axis=1, keepdims=True) + d2b_ref[...]
        o_ref[...] = 1.0 / (1.0 + jnp.exp(-z))


_pk_spec = pl.BlockSpec((PB, LANES), lambda i: (i, 0))
_pk_spec_hi = pl.BlockSpec((PB, LANES), lambda i: (G + i, 0))
_vec_spec = pl.BlockSpec((1, QQ, LANES), lambda i: (i, 0, 0))
_wb_spec = pl.BlockSpec((LANES, LANES), lambda i: (0, 0))


def _tc_prep(x_v, dego0, dego1):
    return pl.pallas_call(
        _tc_prep_body,
        grid=(G,),
        in_specs=[_pk_spec, _vec_spec, _vec_spec],
        out_specs=_pk_spec,
        out_shape=jax.ShapeDtypeStruct((PR, LANES), _f32),
    )(x_v, dego0, dego1)


def _tc_combine1(agg_pk, ycur, degi0, degi1, dego0, dego1, wb):
    return pl.pallas_call(
        _tc_combine1_body,
        grid=(G,),
        in_specs=[_pk_spec, _pk_spec_hi, _pk_spec,
                  _vec_spec, _vec_spec, _vec_spec, _vec_spec, _wb_spec],
        out_specs=_pk_spec,
        out_shape=jax.ShapeDtypeStruct((PR, LANES), _f32),
    )(agg_pk, agg_pk, ycur, degi0, degi1, dego0, dego1, wb)


def _tc_combine2(agg_pk, ycur, y1, degi0, degi1, dego0, dego1, wb,
                 d1w, d1b, d2wt, d2b):
    return pl.pallas_call(
        _tc_combine2_body,
        grid=(G,),
        in_specs=[_pk_spec, _pk_spec_hi, _pk_spec, _pk_spec,
                  _vec_spec, _vec_spec, _vec_spec, _vec_spec, _wb_spec,
                  pl.BlockSpec((D, PRED_H), lambda i: (0, 0)),
                  pl.BlockSpec((1, PRED_H), lambda i: (0, 0)),
                  pl.BlockSpec((1, PRED_H), lambda i: (0, 0)),
                  pl.BlockSpec((1, 1), lambda i: (0, 0))],
        out_specs=pl.BlockSpec((1, 1), lambda i: (0, 0)),
        out_shape=jax.ShapeDtypeStruct((1, 1), _f32),
        scratch_shapes=[pltpu.VMEM((1, LANES), _f32)],
    )(agg_pk, agg_pk, ycur, y1, degi0, degi1, dego0, dego1, wb,
      d1w, d1b, d2wt, d2b)


def kernel(x, edge_index, w1, w2, dec1_w, dec1_b, dec2_w, dec2_b):
    src_p = edge_index[0].reshape(EROWS, LANES)
    dst_p = edge_index[1].reshape(EROWS, LANES)
    x_v = x.reshape(PRX, LANES)
    eye8 = jnp.eye(8, dtype=_f32)
    w1b = jnp.kron(eye8, w1)
    w2b = jnp.kron(eye8, w2)

    sc_degrees, sc_message = _sc_kernels()
    dego_f, degi_f = sc_degrees(src_p, dst_p)
    dego0 = dego_f[:NP].reshape(G, QQ, LANES)
    dego1 = dego_f[NP:].reshape(G, QQ, LANES)
    degi0 = degi_f[:NP].reshape(G, QQ, LANES)
    degi1 = degi_f[NP:].reshape(G, QQ, LANES)

    y1 = _tc_prep(x_v, dego0, dego1)

    agg1 = sc_message(y1.reshape(NP, D), src_p, dst_p)
    agg1_pk = agg1.reshape(2 * PR, LANES)

    y2 = _tc_combine1(agg1_pk, y1, degi0, degi1, dego0, dego1, w1b)

    agg2 = sc_message(y2.reshape(NP, D), src_p, dst_p)
    agg2_pk = agg2.reshape(2 * PR, LANES)

    o = _tc_combine2(agg2_pk, y2, y1, degi0, degi1, dego0, dego1, w2b,
                     dec1_w, dec1_b.reshape(1, PRED_H),
                     dec2_w.reshape(1, PRED_H), dec2_b.reshape(1, 1))
    return o


# speedup vs baseline: 108.5518x; 1.0875x over previous
import functools

import jax
import jax.numpy as jnp
import numpy as _np
from jax import lax
from jax.experimental import pallas as pl
from jax.experimental.pallas import tpu as pltpu
from jax.experimental.pallas import tpu_sc as plsc

N = 100000
E = 3200000
D = 16
PRED_H = 32
ALPHA = 0.5
BETA1 = float(_np.log(1.0 / 1.0 + 1.0))
BETA2 = float(_np.log(1.0 / 2.0 + 1.0))

NC = 2
NS = 16
NW = NC * NS
LANES = 128

NP = 100352
RPT = NP // NS
EROWS = E // LANES
KG = 4
KD2 = 8

PR = NP // 8
PRX = N * D // LANES
RB = 2048
PB = RB // 8
QQ = RB // LANES
G = NP // RB

_f32 = jnp.float32
_i32 = jnp.int32


def _fill(ref, n, value):
    def body(i, _):
        ref[pl.ds(i * 16, 16)] = jnp.full((16,), value, _f32)
        return 0
    lax.fori_loop(0, n // 16, body, 0)


def _worker_groups(wid, k):
    gtot = EROWS // k
    gbase, grem = gtot // NW, gtot % NW
    extra = jnp.minimum(wid, grem)
    return wid * gbase + extra, gbase + jnp.where(wid < grem, 1, 0)


def _sc_degrees_body(src_hbm, dst_hbm, dego_hbm, degi_hbm,
                     sidx, didx, ones_v, zbuf, dego_s, degi_s, semi, semd):
    cid = lax.axis_index("c")
    sid = lax.axis_index("s")
    wid = sid * NC + cid

    _fill(ones_v, LANES, 1.0)
    _fill(zbuf, RPT, 0.0)
    pltpu.sync_copy(zbuf, dego_s.at[pl.ds(sid * RPT, RPT)])
    pltpu.sync_copy(zbuf, degi_s.at[pl.ds(sid * RPT, RPT)])
    plsc.subcore_barrier()

    g0, ng = _worker_groups(wid, KD2)

    def fire_idx(g):
        base = (g0 + g) * KD2
        pltpu.async_copy(src_hbm.at[pl.ds(base, KD2)], sidx.at[g % 3], semi)
        pltpu.async_copy(dst_hbm.at[pl.ds(base, KD2)], didx.at[g % 3], semi)

    def drain_idx():
        for _ in range(2):
            pltpu.make_async_copy(
                src_hbm.at[pl.ds(0, KD2)], sidx.at[0], semi).wait()

    def drain_scatter():
        for _ in range(2 * KD2):
            pltpu.make_async_copy(
                src_hbm.at[pl.ds(0, 1)], sidx.at[0, pl.ds(0, 1)], semd).wait()

    fire_idx(0)

    def group(g, _):
        b = g % 3

        @pl.when(g >= 2)
        def _():
            drain_scatter()

        drain_idx()

        @pl.when(g + 1 < ng)
        def _():
            fire_idx(g + 1)

        for j in range(KD2):
            pltpu.async_copy(ones_v, dego_s.at[sidx.at[b, j]], semd, add=True)
            pltpu.async_copy(ones_v, degi_s.at[didx.at[b, j]], semd, add=True)
        return 0

    lax.fori_loop(0, ng, group, 0)
    drain_scatter()
    drain_scatter()
    plsc.subcore_barrier()

    off = cid * NP + sid * RPT
    pltpu.sync_copy(dego_s.at[pl.ds(sid * RPT, RPT)], dego_hbm.at[pl.ds(off, RPT)])
    pltpu.sync_copy(degi_s.at[pl.ds(sid * RPT, RPT)], degi_hbm.at[pl.ds(off, RPT)])


def _sc_message_body(y_hbm, src_hbm, dst_hbm, agg_hbm,
                     sidx, didx, rows, agg_s, semi, semg, sems):
    cid = lax.axis_index("c")
    sid = lax.axis_index("s")
    wid = sid * NC + cid
    GROUP_ROWS = KG * LANES

    def zrow(i, _):
        rows[0, i, :] = jnp.zeros((D,), _f32)
        return 0
    lax.fori_loop(0, GROUP_ROWS, zrow, 0)
    r0 = sid * RPT
    for q in range(RPT // GROUP_ROWS):
        pltpu.sync_copy(rows.at[0], agg_s.at[pl.ds(r0 + q * GROUP_ROWS, GROUP_ROWS)])
    rem = RPT % GROUP_ROWS
    if rem:
        pltpu.sync_copy(rows.at[0, pl.ds(0, rem)],
                        agg_s.at[pl.ds(r0 + RPT - rem, rem)])
    plsc.subcore_barrier()

    g0, ng = _worker_groups(wid, KG)

    def fire_idx(g):
        base = (g0 + g) * KG
        pltpu.async_copy(src_hbm.at[pl.ds(base, KG)], sidx.at[g % 4], semi)
        pltpu.async_copy(dst_hbm.at[pl.ds(base, KG)], didx.at[g % 4], semi)

    def drain_idx():
        for _ in range(2):
            pltpu.make_async_copy(
                src_hbm.at[pl.ds(0, KG)], sidx.at[0], semi).wait()

    def fire_gathers(g):
        for j in range(KG):
            pltpu.async_copy(
                y_hbm.at[sidx.at[g % 4, j]],
                rows.at[g % 3, pl.ds(j * LANES, LANES)], semg.at[g % 2])

    def drain_gathers(g):
        for _ in range(KG):
            pltpu.make_async_copy(
                y_hbm.at[pl.ds(0, LANES)], rows.at[0, pl.ds(0, LANES)],
                semg.at[g % 2]).wait()

    def drain_scatter():
        for _ in range(KG):
            pltpu.make_async_copy(
                y_hbm.at[pl.ds(0, LANES)], rows.at[0, pl.ds(0, LANES)],
                sems).wait()

    fire_idx(0)
    fire_idx(1)
    drain_idx()
    fire_gathers(0)

    def group(g, _):
        @pl.when(g >= 2)
        def _():
            drain_scatter()

        @pl.when(g + 1 < ng)
        def _():
            drain_idx()
            @pl.when(g + 2 < ng)
            def _():
                fire_idx(g + 2)
            fire_gathers(g + 1)

        drain_gathers(g)
        for j in range(KG):
            pltpu.async_copy(
                rows.at[g % 3, pl.ds(j * LANES, LANES)],
                agg_s.at[didx.at[g % 4, j]], sems, add=True)
        return 0

    lax.fori_loop(0, ng, group, 0)
    drain_scatter()
    drain_scatter()
    plsc.subcore_barrier()

    off = cid * NP + sid * RPT
    pltpu.sync_copy(agg_s.at[pl.ds(sid * RPT, RPT)], agg_hbm.at[pl.ds(off, RPT)])


@functools.lru_cache(maxsize=None)
def _sc_kernels():
    mesh = plsc.VectorSubcoreMesh(
        core_axis_name="c", subcore_axis_name="s",
        num_cores=NC, num_subcores=NS)
    params = pltpu.CompilerParams(use_tc_tiling_on_sc=False)
    sc_degrees = pl.kernel(
        _sc_degrees_body,
        out_type=(
            jax.ShapeDtypeStruct((NC * NP,), _f32),
            jax.ShapeDtypeStruct((NC * NP,), _f32),
        ),
        mesh=mesh,
        scratch_types=[
            pltpu.VMEM((3, KD2, LANES), _i32),
            pltpu.VMEM((3, KD2, LANES), _i32),
            pltpu.VMEM((LANES,), _f32),
            pltpu.VMEM((RPT,), _f32),
            pltpu.VMEM_SHARED((NP,), _f32),
            pltpu.VMEM_SHARED((NP,), _f32),
            pltpu.SemaphoreType.DMA,
            pltpu.SemaphoreType.DMA,
        ],
        compiler_params=params,
    )
    sc_message = pl.kernel(
        _sc_message_body,
        out_type=jax.ShapeDtypeStruct((NC * NP, D), _f32),
        mesh=mesh,
        scratch_types=[
            pltpu.VMEM((4, KG, LANES), _i32),
            pltpu.VMEM((4, KG, LANES), _i32),
            pltpu.VMEM((3, KG * LANES, D), _f32),
            pltpu.VMEM_SHARED((NP, D), _f32),
            pltpu.SemaphoreType.DMA,
            pltpu.SemaphoreType.DMA((2,)),
            pltpu.SemaphoreType.DMA,
        ],
        compiler_params=params,
    )
    return sc_degrees, sc_message


def _scale_packed(s8):
    rr = lax.broadcasted_iota(_i32, (PB, QQ), 0)
    qq = lax.broadcasted_iota(_i32, (PB, QQ), 1)
    fold8 = (rr // D == qq).astype(_f32)
    s_exp = jnp.dot(fold8, s8, preferred_element_type=_f32)
    tt = lax.broadcasted_iota(_i32, (PB, LANES), 0)
    mm = lax.broadcasted_iota(_i32, (PB, LANES), 1)
    sell = (mm // 8 == tt % D).astype(_f32)
    m2 = lax.broadcasted_iota(_i32, (LANES, LANES), 0)
    c2 = lax.broadcasted_iota(_i32, (LANES, LANES), 1)
    selr = (c2 // D == m2 % 8).astype(_f32)
    return jnp.dot(sell * s_exp, selr, preferred_element_type=_f32)


def _tc_prep_body(x_ref, do0_ref, do1_ref, y_ref):
    i = pl.program_id(0)
    s_out = lax.rsqrt(do0_ref[0] + do1_ref[0] + 1.0)
    y_ref[...] = x_ref[...] * _scale_packed(s_out)

    @pl.when(i == G - 1)
    def _():
        y_ref[pl.ds(PRX - (G - 1) * PB, PB * G - PRX), :] = jnp.zeros(
            (PB * G - PRX, LANES), _f32)


def _tc_combine1_body(p0_ref, p1_ref, yc_ref,
                      di0_ref, di1_ref, do0_ref, do1_ref, w_ref, y2_ref):
    do8 = do0_ref[0] + do1_ref[0] + 1.0
    s_in = lax.rsqrt(di0_ref[0] + di1_ref[0] + 1.0)
    yc = yc_ref[...]
    agg = (p0_ref[...] + p1_ref[...] + yc) * _scale_packed(s_in)
    h = (1.0 - ALPHA) * agg + ALPHA * (yc * _scale_packed(lax.sqrt(do8)))
    hw = jnp.dot(h, w_ref[...], preferred_element_type=_f32)
    x1 = jnp.maximum((1.0 - BETA1) * h + BETA1 * hw, 0.0)
    y2_ref[...] = x1 * _scale_packed(lax.rsqrt(do8))


def _tc_combine2_body(p0_ref, p1_ref, yc_ref, y1_ref,
                      di0_ref, di1_ref, do0_ref, do1_ref, w_ref,
                      d1w_ref, d1b_ref, d2wt_ref, d2b_ref, o_ref, acc_ref):
    i = pl.program_id(0)

    @pl.when(i == 0)
    def _():
        acc_ref[...] = jnp.zeros((1, LANES), _f32)

    do8 = do0_ref[0] + do1_ref[0] + 1.0
    s_in = lax.rsqrt(di0_ref[0] + di1_ref[0] + 1.0)
    yc = yc_ref[...]
    agg = (p0_ref[...] + p1_ref[...] + yc) * _scale_packed(s_in)
    h = (1.0 - ALPHA) * agg + ALPHA * (y1_ref[...] * _scale_packed(lax.sqrt(do8)))
    hw = jnp.dot(h, w_ref[...], preferred_element_type=_f32)
    x2 = jnp.maximum((1.0 - BETA2) * h + BETA2 * hw, 0.0)
    acc_ref[...] += jnp.sum(x2, axis=0, keepdims=True)

    @pl.when(i == G - 1)
    def _():
        c2 = lax.broadcasted_iota(_i32, (LANES, D), 0)
        f2 = lax.broadcasted_iota(_i32, (LANES, D), 1)
        fold = (c2 % D == f2).astype(_f32)
        pooled = jnp.dot(acc_ref[...], fold,
                         preferred_element_type=_f32) * (1.0 / N)
        hm = jnp.dot(pooled, d1w_ref[...], preferred_element_type=_f32)
        hm = jnp.maximum(hm + d1b_ref[...], 0.0)
        z = jnp.sum(hm * d2wt_ref[...], axis=1, keepdims=True) + d2b_ref[...]
        o_ref[...] = 1.0 / (1.0 + jnp.exp(-z))


_pk_spec = pl.BlockSpec((PB, LANES), lambda i: (i, 0))
_pk_spec_hi = pl.BlockSpec((PB, LANES), lambda i: (G + i, 0))
_vec_spec = pl.BlockSpec((1, QQ, LANES), lambda i: (i, 0, 0))
_wb_spec = pl.BlockSpec((LANES, LANES), lambda i: (0, 0))


def _tc_prep(x_v, dego0, dego1):
    return pl.pallas_call(
        _tc_prep_body,
        grid=(G,),
        in_specs=[_pk_spec, _vec_spec, _vec_spec],
        out_specs=_pk_spec,
        out_shape=jax.ShapeDtypeStruct((PR, LANES), _f32),
    )(x_v, dego0, dego1)


def _tc_combine1(agg_pk, ycur, degi0, degi1, dego0, dego1, wb):
    return pl.pallas_call(
        _tc_combine1_body,
        grid=(G,),
        in_specs=[_pk_spec, _pk_spec_hi, _pk_spec,
                  _vec_spec, _vec_spec, _vec_spec, _vec_spec, _wb_spec],
        out_specs=_pk_spec,
        out_shape=jax.ShapeDtypeStruct((PR, LANES), _f32),
    )(agg_pk, agg_pk, ycur, degi0, degi1, dego0, dego1, wb)


def _tc_combine2(agg_pk, ycur, y1, degi0, degi1, dego0, dego1, wb,
                 d1w, d1b, d2wt, d2b):
    return pl.pallas_call(
        _tc_combine2_body,
        grid=(G,),
        in_specs=[_pk_spec, _pk_spec_hi, _pk_spec, _pk_spec,
                  _vec_spec, _vec_spec, _vec_spec, _vec_spec, _wb_spec,
                  pl.BlockSpec((D, PRED_H), lambda i: (0, 0)),
                  pl.BlockSpec((1, PRED_H), lambda i: (0, 0)),
                  pl.BlockSpec((1, PRED_H), lambda i: (0, 0)),
                  pl.BlockSpec((1, 1), lambda i: (0, 0))],
        out_specs=pl.BlockSpec((1, 1), lambda i: (0, 0)),
        out_shape=jax.ShapeDtypeStruct((1, 1), _f32),
        scratch_shapes=[pltpu.VMEM((1, LANES), _f32)],
    )(agg_pk, agg_pk, ycur, y1, degi0, degi1, dego0, dego1, wb,
      d1w, d1b, d2wt, d2b)


def kernel(x, edge_index, w1, w2, dec1_w, dec1_b, dec2_w, dec2_b):
    src_p = edge_index[0].reshape(EROWS, LANES)
    dst_p = edge_index[1].reshape(EROWS, LANES)
    x_v = x.reshape(PRX, LANES)
    eye8 = jnp.eye(8, dtype=_f32)
    w1b = jnp.kron(eye8, w1)
    w2b = jnp.kron(eye8, w2)

    sc_degrees, sc_message = _sc_kernels()
    dego_f, degi_f = sc_degrees(src_p, dst_p)
    dego0 = dego_f[:NP].reshape(G, QQ, LANES)
    dego1 = dego_f[NP:].reshape(G, QQ, LANES)
    degi0 = degi_f[:NP].reshape(G, QQ, LANES)
    degi1 = degi_f[NP:].reshape(G, QQ, LANES)

    y1 = _tc_prep(x_v, dego0, dego1)

    agg1 = sc_message(y1.reshape(NP, D), src_p, dst_p)
    agg1_pk = agg1.reshape(2 * PR, LANES)

    y2 = _tc_combine1(agg1_pk, y1, degi0, degi1, dego0, dego1, w1b)

    agg2 = sc_message(y2.reshape(NP, D), src_p, dst_p)
    agg2_pk = agg2.reshape(2 * PR, LANES)

    o = _tc_combine2(agg2_pk, y2, y1, degi0, degi1, dego0, dego1, w2b,
                     dec1_w, dec1_b.reshape(1, PRED_H),
                     dec2_w.reshape(1, PRED_H), dec2_b.reshape(1, 1))
    return o
